# baseline (speedup 1.0000x reference)
import numpy as np
import jax
import jax.numpy as jnp
from jax.experimental import pallas as pl

_AVG_LOG = float(np.mean(np.log(np.arange(1, 31, dtype=np.float64))))
_N = 10000


def _copy_body(x_ref, o_ref):
    o_ref[...] = x_ref[...]


def _pna(x, src, dst, edge_attr, We, be, Wpre, bpre, Wpost, bpost, Wlin, blin):
    e = edge_attr @ We + be
    h = jnp.concatenate([x[dst], x[src], e], axis=-1)
    m = h @ Wpre + bpre
    ones = jnp.ones((src.shape[0],), dtype=x.dtype)
    cnt = jax.ops.segment_sum(ones, dst, num_segments=_N)
    cntc = jnp.clip(cnt, 1.0, None)[:, None]
    mean = jax.ops.segment_sum(m, dst, num_segments=_N) / cntc
    mean_sq = jax.ops.segment_sum(m * m, dst, num_segments=_N) / cntc
    std = jnp.sqrt(jax.nn.relu(mean_sq - mean * mean) + 1e-5)
    mx = jnp.where(cnt[:, None] > 0, jax.ops.segment_max(m, dst, num_segments=_N), 0.0)
    mn = jnp.where(cnt[:, None] > 0, -jax.ops.segment_max(-m, dst, num_segments=_N), 0.0)
    agg = jnp.concatenate([mean, mx, mn, std], axis=-1)
    amp = jnp.log(cntc + 1.0) / _AVG_LOG
    att = _AVG_LOG / jnp.log(cntc + 1.0)
    out = jnp.concatenate([x, agg, agg * amp, agg * att], axis=-1)
    out = out @ Wpost + bpost
    return out @ Wlin + blin


def _bn(h, g, b):
    mu = jnp.mean(h, axis=0)
    var = jnp.var(h, axis=0)
    return g * (h - mu) / jnp.sqrt(var + 1e-5) + b


def kernel(x, edge_index, edge_attr, We1, be1, Wpre1, bpre1, Wpost1, bpost1, Wlin1, blin1, g1, bb1, We2, be2, Wpre2, bpre2, Wpost2, bpost2, Wlin2, blin2, g2, bb2, Wout, bout):
    x = pl.pallas_call(
        _copy_body,
        out_shape=jax.ShapeDtypeStruct(x.shape, x.dtype),
    )(x)
    src = edge_index[0]
    dst = edge_index[1]
    h = _pna(x, src, dst, edge_attr, We1, be1, Wpre1, bpre1, Wpost1, bpost1, Wlin1, blin1)
    h = jax.nn.elu(_bn(h, g1, bb1))
    h = _pna(h, src, dst, edge_attr, We2, be2, Wpre2, bpre2, Wpost2, bpost2, Wlin2, blin2)
    h = jax.nn.elu(_bn(h, g2, bb2))
    return h @ Wout + bout


# speedup vs baseline: 1.7953x; 1.7953x over previous
import functools
import numpy as np
import jax
import jax.numpy as jnp
from jax import lax
from jax.experimental import pallas as pl
from jax.experimental.pallas import tpu as pltpu
from jax.experimental.pallas import tpu_sc as plsc

AVG_LOG = float(np.mean(np.log(np.arange(1, 31, dtype=np.float64))))
F32 = jnp.float32

N = 10000
E = 160000
NW = 32
R = 320
NP = NW * R
NQ = 4
QW = 64
K = 128
CH = 1600
FL = 2048
ECAP = E + FL

_mesh = plsc.VectorSubcoreMesh(core_axis_name="c", subcore_axis_name="s",
                               num_cores=2, num_subcores=16)


def _wid():
    return lax.axis_index("s") * 2 + lax.axis_index("c")


def _bin_body(dst_hbm, src_hbm,
              eids_hbm, gdst_hbm, gsrc_hbm, counts_hbm, cnt_hbm,
              dbuf, sbuf, st_e, st_d, st_s, cacc, cvec):
    w = _wid()
    lo = w * R
    iota = lax.iota(jnp.int32, 16)
    ones = jnp.ones((16,), F32)
    zeros = jnp.zeros((16,), F32)

    def initc(t, _):
        cacc[pl.ds(t * 16, 16)] = zeros
        return 0
    lax.fori_loop(0, R // 16, initc, 0)

    def chunk(ci, carry):
        pltpu.sync_copy(dst_hbm.at[pl.ds(ci * CH, CH)], dbuf)
        pltpu.sync_copy(src_hbm.at[pl.ds(ci * CH, CH)], sbuf)

        def vec(vi, c2):
            wp, off = c2
            d16 = dbuf[pl.ds(vi * 16, 16)]
            s16 = sbuf[pl.ds(vi * 16, 16)]
            eid = ci * CH + vi * 16 + iota
            m = (d16 >= lo) & (d16 < lo + R)
            ld = jnp.clip(d16 - lo, 0, R - 1)
            plsc.addupdate_scatter(cacc, [ld], ones, mask=m)
            plsc.store_compressed(st_e.at[pl.ds(wp, 16)], eid, mask=m)
            plsc.store_compressed(st_d.at[pl.ds(wp, 16)], d16, mask=m)
            plsc.store_compressed(st_s.at[pl.ds(wp, 16)], s16, mask=m)
            npop = plsc.all_reduce_population_count(m)
            if npop.ndim:
                npop = npop[0]
            wp2 = wp + npop
            fl = wp2 >= FL

            @pl.when(fl)
            def _flush():
                pltpu.sync_copy(st_e.at[pl.ds(0, FL)],
                                eids_hbm.at[pl.ds(pl.multiple_of(w * ECAP + off, 8), FL)])
                pltpu.sync_copy(st_d.at[pl.ds(0, FL)],
                                gdst_hbm.at[pl.ds(pl.multiple_of(w * ECAP + off, 8), FL)])
                pltpu.sync_copy(st_s.at[pl.ds(0, FL)],
                                gsrc_hbm.at[pl.ds(pl.multiple_of(w * ECAP + off, 8), FL)])
                st_e[pl.ds(0, 16)] = st_e[pl.ds(FL, 16)]
                st_d[pl.ds(0, 16)] = st_d[pl.ds(FL, 16)]
                st_s[pl.ds(0, 16)] = st_s[pl.ds(FL, 16)]

            wp3 = jnp.where(fl, wp2 - FL, wp2)
            off2 = jnp.where(fl, off + FL, off)
            return wp3, off2

        return lax.fori_loop(0, CH // 16, vec, carry)

    wp, off = lax.fori_loop(0, E // CH, chunk,
                            (jnp.int32(0), jnp.int32(0)))
    pltpu.sync_copy(st_e.at[pl.ds(0, FL)],
                    eids_hbm.at[pl.ds(pl.multiple_of(w * ECAP + off, 8), FL)])
    pltpu.sync_copy(st_d.at[pl.ds(0, FL)],
                    gdst_hbm.at[pl.ds(pl.multiple_of(w * ECAP + off, 8), FL)])
    pltpu.sync_copy(st_s.at[pl.ds(0, FL)],
                    gsrc_hbm.at[pl.ds(pl.multiple_of(w * ECAP + off, 8), FL)])
    cvec[...] = jnp.full((16,), off + wp, jnp.int32)
    pltpu.sync_copy(cvec.at[pl.ds(0, 8)], counts_hbm.at[pl.ds(pl.multiple_of(w * 8, 8), 8)])
    pltpu.sync_copy(cacc, cnt_hbm.at[pl.ds(pl.multiple_of(lo, 8), R)])


def _binning(dst, src):
    f = pl.kernel(
        _bin_body,
        compiler_params=pltpu.CompilerParams(needs_layout_passes=False, use_tc_tiling_on_sc=False),
        out_type=(
            jax.ShapeDtypeStruct((NW * ECAP,), jnp.int32),
            jax.ShapeDtypeStruct((NW * ECAP,), jnp.int32),
            jax.ShapeDtypeStruct((NW * ECAP,), jnp.int32),
            jax.ShapeDtypeStruct((NW * 8,), jnp.int32),
            jax.ShapeDtypeStruct((NP,), F32),
        ),
        mesh=_mesh,
        scratch_types=[
            pltpu.VMEM((CH,), jnp.int32),
            pltpu.VMEM((CH,), jnp.int32),
            pltpu.VMEM((FL + 16,), jnp.int32),
            pltpu.VMEM((FL + 16,), jnp.int32),
            pltpu.VMEM((FL + 16,), jnp.int32),
            pltpu.VMEM((R,), F32),
            pltpu.VMEM((16,), jnp.int32),
        ],
    )
    return f(dst, src)


def _scmain_body(eids_hbm, gdst_hbm, gsrc_hbm, counts_hbm, xd_hbm, xs_hbm,
                 ee_hbm,
                 sum_hbm, sq_hbm, mx_hbm, mn_hbm,
                 cbuf, xdloc, rxs, ree, ev, dv, sv, ixs, iee,
                 acc_s, acc_q, acc_mx, acc_mn, sem0, sem1):
    w = _wid()
    lo = w * R
    iota = lax.iota(jnp.int32, 16)
    zeros = jnp.zeros((16,), F32)
    neg = jnp.full((16,), -3.0e38, F32)
    pos = jnp.full((16,), 3.0e38, F32)

    pltpu.sync_copy(counts_hbm, cbuf.at[pl.ds(0, NW * 8)])
    cw = cbuf[pl.ds(w * 8, 16)][0]
    nblk = lax.div(cw + (K - 1), jnp.int32(K))

    for q in range(NQ):
        def initrow(r, _):
            for v in range(QW // 16):
                cs = pl.ds(v * 16, 16)
                acc_s[r, cs] = zeros
                acc_q[r, cs] = zeros
                acc_mx[r, cs] = neg
                acc_mn[r, cs] = pos
            return 0
        lax.fori_loop(0, R, initrow, 0)

        pltpu.sync_copy(xd_hbm.at[pl.ds(pl.multiple_of(q * NP + lo, 8), R)], xdloc)

        def blk(b, _):
            base = b * K
            pltpu.sync_copy(eids_hbm.at[pl.ds(pl.multiple_of(w * ECAP + base, 8), K)], ev)
            pltpu.sync_copy(gdst_hbm.at[pl.ds(pl.multiple_of(w * ECAP + base, 8), K)],
                            dv.at[pl.ds(0, K)])
            pltpu.sync_copy(gsrc_hbm.at[pl.ds(pl.multiple_of(w * ECAP + base, 8), K)], sv)
            nval = jnp.minimum(cw - base, K)
            for t in range(K // 16):
                mk = (t * 16 + iota) < nval
                e16 = jnp.where(mk, ev[pl.ds(t * 16, 16)], 0)
                s16 = jnp.where(mk, sv[pl.ds(t * 16, 16)], 0)
                iee[pl.ds(t * 16, 16)] = e16 + q * E
                ixs[pl.ds(t * 16, 16)] = s16 + q * NP
            c0 = pltpu.async_copy(xs_hbm.at[ixs], rxs, sem0)
            c1 = pltpu.async_copy(ee_hbm.at[iee], ree, sem1)
            c0.wait()
            c1.wait()

            def edge(j, _):
                r = dv[pl.ds(j, 16)][0] - lo
                for v in range(QW // 16):
                    cs = pl.ds(v * 16, 16)
                    mv = xdloc[r, cs] + rxs[j, cs] + ree[j, cs]
                    plsc.addupdate(acc_s.at[r, cs], mv)
                    plsc.addupdate(acc_q.at[r, cs], mv * mv)
                    acc_mx[r, cs] = jnp.maximum(acc_mx[r, cs], mv)
                    acc_mn[r, cs] = jnp.minimum(acc_mn[r, cs], mv)
                return 0
            lax.fori_loop(0, nval, edge, 0)
            return 0

        lax.fori_loop(0, nblk, blk, 0)

        pltpu.sync_copy(acc_s, sum_hbm.at[pl.ds(pl.multiple_of(q * NP + lo, 8), R)])
        pltpu.sync_copy(acc_q, sq_hbm.at[pl.ds(pl.multiple_of(q * NP + lo, 8), R)])
        pltpu.sync_copy(acc_mx, mx_hbm.at[pl.ds(pl.multiple_of(q * NP + lo, 8), R)])
        pltpu.sync_copy(acc_mn, mn_hbm.at[pl.ds(pl.multiple_of(q * NP + lo, 8), R)])


def _scmain(eids, gdst, gsrc, counts, xd, xs, ee):
    f = pl.kernel(
        _scmain_body,
        compiler_params=pltpu.CompilerParams(needs_layout_passes=False, use_tc_tiling_on_sc=False),
        out_type=(
            jax.ShapeDtypeStruct((NQ * NP, QW), F32),
            jax.ShapeDtypeStruct((NQ * NP, QW), F32),
            jax.ShapeDtypeStruct((NQ * NP, QW), F32),
            jax.ShapeDtypeStruct((NQ * NP, QW), F32),
        ),
        mesh=_mesh,
        scratch_types=[
            pltpu.VMEM((NW * 8 + 16,), jnp.int32),
            pltpu.VMEM((R, QW), F32),
            pltpu.VMEM((K, QW), F32),
            pltpu.VMEM((K, QW), F32),
            pltpu.VMEM((K,), jnp.int32),
            pltpu.VMEM((K + 16,), jnp.int32),
            pltpu.VMEM((K,), jnp.int32),
            pltpu.VMEM((K,), jnp.int32),
            pltpu.VMEM((K,), jnp.int32),
            pltpu.VMEM((R, QW), F32),
            pltpu.VMEM((R, QW), F32),
            pltpu.VMEM((R, QW), F32),
            pltpu.VMEM((R, QW), F32),
            pltpu.SemaphoreType.DMA,
            pltpu.SemaphoreType.DMA,
        ],
    )
    return f(eids, gdst, gsrc, counts, xd, xs, ee)


def _wprep_body(We_ref, Wpe_ref, be_ref, bpre_ref, Wc_ref, cb_ref):
    Wc_ref[...] = jnp.dot(We_ref[...], Wpe_ref[...],
                          preferred_element_type=F32)
    cb_ref[...] = jnp.dot(be_ref[...], Wpe_ref[...],
                          preferred_element_type=F32) + bpre_ref[...]


def _wprep(We, Wpe, be, bpre):
    de, d = We.shape
    return pl.pallas_call(
        _wprep_body,
        out_shape=(jax.ShapeDtypeStruct((de, d), F32),
                   jax.ShapeDtypeStruct((1, d), F32)),
    )(We, Wpe, be.reshape(1, -1), bpre.reshape(1, -1))


def _xprep_body(x_ref, Wd_ref, Ws_ref, xd_ref, xs_ref):
    xb = x_ref[...]
    xd_ref[...] = jnp.dot(xb, Wd_ref[0], preferred_element_type=F32)
    xs_ref[...] = jnp.dot(xb, Ws_ref[0], preferred_element_type=F32)


def _xprep(x, Wd4, Ws4):
    d = x.shape[1]
    nb = NP // R
    return pl.pallas_call(
        _xprep_body,
        grid=(nb, NQ),
        in_specs=[
            pl.BlockSpec((R, d), lambda i, q: (i, 0)),
            pl.BlockSpec((1, d, QW), lambda i, q: (q, 0, 0)),
            pl.BlockSpec((1, d, QW), lambda i, q: (q, 0, 0)),
        ],
        out_specs=(
            pl.BlockSpec((R, QW), lambda i, q: (q * nb + i, 0)),
            pl.BlockSpec((R, QW), lambda i, q: (q * nb + i, 0)),
        ),
        out_shape=(jax.ShapeDtypeStruct((NQ * NP, QW), F32),
                   jax.ShapeDtypeStruct((NQ * NP, QW), F32)),
    )(x, Wd4, Ws4)


def _eeprep_body(ea_ref, Wc_ref, cb_ref, ee_ref):
    ee_ref[...] = jnp.dot(ea_ref[...], Wc_ref[0],
                          preferred_element_type=F32) + cb_ref[0]


def _eeprep(ea, Wc, cb):
    de = ea.shape[1]
    eb = 640
    nb = E // eb
    Wc4 = Wc.reshape(de, NQ, QW).transpose(1, 0, 2)
    cb4 = cb.reshape(1, NQ, QW).transpose(1, 0, 2)
    return pl.pallas_call(
        _eeprep_body,
        grid=(nb, NQ),
        in_specs=[
            pl.BlockSpec((eb, de), lambda i, q: (i, 0)),
            pl.BlockSpec((1, de, QW), lambda i, q: (q, 0, 0)),
            pl.BlockSpec((1, 1, QW), lambda i, q: (q, 0, 0)),
        ],
        out_specs=pl.BlockSpec((eb, QW), lambda i, q: (q * nb + i, 0)),
        out_shape=jax.ShapeDtypeStruct((NQ * E, QW), F32),
    )(ea, Wc4, cb4)


def _post_body(x_ref, cnt_ref, s_ref, q_ref, mx_ref, mn_ref,
               Wa_ref, Wb_ref, Wc_ref, bpost_ref, Wlin_ref, blin_ref,
               h_ref, st_ref):
    i = pl.program_id(0)
    cnt = cnt_ref[...]
    cntc = jnp.maximum(cnt, 1.0)
    rc = 1.0 / cntc
    posm = cnt > 0.0
    parts = []
    for q in range(NQ):
        sm = s_ref[q]
        sq = q_ref[q]
        mean = sm * rc
        msq = sq * rc
        std = jnp.sqrt(jnp.maximum(msq - mean * mean, 0.0) + 1e-5)
        mx = jnp.where(posm, mx_ref[q], 0.0)
        mn = jnp.where(posm, mn_ref[q], 0.0)
        parts.append((mean, mx, mn, std))
    agg = jnp.concatenate(
        [parts[q][a] for a in range(4) for q in range(NQ)], axis=1)
    lc = jnp.log(cntc + 1.0)
    amp = lc * (1.0 / AVG_LOG)
    att = AVG_LOG / lc
    cat = jnp.concatenate([x_ref[...], agg], axis=1)
    P = jnp.dot(cat, Wa_ref[...], preferred_element_type=F32) + bpost_ref[...]
    P = P + amp * jnp.dot(agg, Wb_ref[...], preferred_element_type=F32)
    P = P + att * jnp.dot(agg, Wc_ref[...], preferred_element_type=F32)
    H = jnp.dot(P, Wlin_ref[...], preferred_element_type=F32) + blin_ref[...]
    h_ref[...] = H
    cs = jnp.sum(H, axis=0, keepdims=True)
    cq = jnp.sum(H * H, axis=0, keepdims=True)
    st = jnp.concatenate(
        [cs, cq, jnp.zeros((6, cs.shape[1]), F32)], axis=0)

    @pl.when(i == 0)
    def _():
        st_ref[...] = st

    @pl.when(i > 0)
    def _():
        st_ref[...] = st_ref[...] + st


def _post(x, cnt, s4, q4, mx4, mn4, Wa, Wb, Wc, bpost, Wlin, blin):
    d = x.shape[1]
    h = Wlin.shape[1]
    rb = 1000
    nb = N // rb
    return pl.pallas_call(
        _post_body,
        grid=(nb,),
        in_specs=[
            pl.BlockSpec((rb, d), lambda i: (i, 0)),
            pl.BlockSpec((rb, 1), lambda i: (i, 0)),
            pl.BlockSpec((NQ, rb, QW), lambda i: (0, i, 0)),
            pl.BlockSpec((NQ, rb, QW), lambda i: (0, i, 0)),
            pl.BlockSpec((NQ, rb, QW), lambda i: (0, i, 0)),
            pl.BlockSpec((NQ, rb, QW), lambda i: (0, i, 0)),
            pl.BlockSpec(Wa.shape, lambda i: (0, 0)),
            pl.BlockSpec(Wb.shape, lambda i: (0, 0)),
            pl.BlockSpec(Wc.shape, lambda i: (0, 0)),
            pl.BlockSpec((1, h), lambda i: (0, 0)),
            pl.BlockSpec((d, h) if Wlin.shape[0] == d else Wlin.shape,
                         lambda i: (0, 0)),
            pl.BlockSpec((1, h), lambda i: (0, 0)),
        ],
        out_specs=(
            pl.BlockSpec((rb, h), lambda i: (i, 0)),
            pl.BlockSpec((8, h), lambda i: (0, 0)),
        ),
        out_shape=(jax.ShapeDtypeStruct((N, h), F32),
                   jax.ShapeDtypeStruct((8, h), F32)),
    )(x, cnt, s4, q4, mx4, mn4, Wa, Wb, Wc, bpost, Wlin, blin)


def _bn_body(h_ref, st_ref, g_ref, b_ref, o_ref):
    mu = st_ref[0:1] * (1.0 / N)
    var = st_ref[1:2] * (1.0 / N) - mu * mu
    y = g_ref[...] * (h_ref[...] - mu) / jnp.sqrt(var + 1e-5) + b_ref[...]
    o_ref[...] = jnp.where(y > 0, y, jnp.exp(y) - 1.0)


def _bn_apply(H, stats, g, b):
    h = H.shape[1]
    rb = 1000
    return pl.pallas_call(
        _bn_body,
        grid=(N // rb,),
        in_specs=[
            pl.BlockSpec((rb, h), lambda i: (i, 0)),
            pl.BlockSpec((8, h), lambda i: (0, 0)),
            pl.BlockSpec((1, h), lambda i: (0, 0)),
            pl.BlockSpec((1, h), lambda i: (0, 0)),
        ],
        out_specs=pl.BlockSpec((rb, h), lambda i: (i, 0)),
        out_shape=jax.ShapeDtypeStruct((N, h), F32),
    )(H, stats, g.reshape(1, -1), b.reshape(1, -1))


def _bn_head_body(h_ref, st_ref, g_ref, b_ref, Wo_ref, bo_ref, o_ref):
    mu = st_ref[0:1] * (1.0 / N)
    var = st_ref[1:2] * (1.0 / N) - mu * mu
    y = g_ref[...] * (h_ref[...] - mu) / jnp.sqrt(var + 1e-5) + b_ref[...]
    y = jnp.where(y > 0, y, jnp.exp(y) - 1.0)
    o_ref[...] = jnp.dot(y, Wo_ref[...], preferred_element_type=F32) \
        + bo_ref[...]


def _bn_head(H, stats, g, b, Wout, bout):
    h = H.shape[1]
    rb = 1000
    return pl.pallas_call(
        _bn_head_body,
        grid=(N // rb,),
        in_specs=[
            pl.BlockSpec((rb, h), lambda i: (i, 0)),
            pl.BlockSpec((8, h), lambda i: (0, 0)),
            pl.BlockSpec((1, h), lambda i: (0, 0)),
            pl.BlockSpec((1, h), lambda i: (0, 0)),
            pl.BlockSpec((h, 1), lambda i: (0, 0)),
            pl.BlockSpec((1, 1), lambda i: (0, 0)),
        ],
        out_specs=pl.BlockSpec((rb, 1), lambda i: (i, 0)),
        out_shape=jax.ShapeDtypeStruct((N, 1), F32),
    )(H, stats, g.reshape(1, -1), b.reshape(1, -1), Wout,
      bout.reshape(1, 1))


def _layer(x, ea, bins, We, be, Wpre, bpre, Wpost, bpost, Wlin, blin):
    eids, gdst, gsrc, counts, cnt = bins
    d = x.shape[1]
    ho = Wpost.shape[1]
    Wd, Ws, Wpe = Wpre[:d], Wpre[d:2 * d], Wpre[2 * d:]
    Wd4 = Wd.reshape(d, NQ, QW).transpose(1, 0, 2)
    Ws4 = Ws.reshape(d, NQ, QW).transpose(1, 0, 2)
    Wc, cb = _wprep(We, Wpe, be, bpre)
    xd, xs = _xprep(x, Wd4, Ws4)
    ee = _eeprep(ea, Wc, cb)
    s4, q4, mx4, mn4 = _scmain(eids, gdst, gsrc, counts, xd, xs, ee)
    s4 = s4.reshape(NQ, NP, QW)
    q4 = q4.reshape(NQ, NP, QW)
    mx4 = mx4.reshape(NQ, NP, QW)
    mn4 = mn4.reshape(NQ, NP, QW)
    cnt2 = cnt[:N].reshape(N, 1)
    Wa = Wpost[:d + 4 * d]
    Wb = Wpost[d + 4 * d:d + 8 * d]
    Wcg = Wpost[d + 8 * d:d + 12 * d]
    return _post(x, cnt2, s4, q4, mx4, mn4, Wa, Wb, Wcg,
                 bpost.reshape(1, -1), Wlin, blin.reshape(1, -1))


def kernel(x, edge_index, edge_attr, We1, be1, Wpre1, bpre1, Wpost1, bpost1,
           Wlin1, blin1, g1, bb1, We2, be2, Wpre2, bpre2, Wpost2, bpost2,
           Wlin2, blin2, g2, bb2, Wout, bout):
    src = edge_index[0]
    dst = edge_index[1]
    bins = _binning(dst, src)
    H1, st1 = _layer(x, edge_attr, bins, We1, be1, Wpre1, bpre1,
                     Wpost1, bpost1, Wlin1, blin1)
    h1 = _bn_apply(H1, st1, g1, bb1)
    H2, st2 = _layer(h1, edge_attr, bins, We2, be2, Wpre2, bpre2,
                     Wpost2, bpost2, Wlin2, blin2)
    return _bn_head(H2, st2, g2, bb2, Wout, bout)


# speedup vs baseline: 2.2498x; 1.2531x over previous
import numpy as np
import jax
import jax.numpy as jnp
from jax import lax
from jax.experimental import pallas as pl
from jax.experimental.pallas import tpu as pltpu
from jax.experimental.pallas import tpu_sc as plsc

AVG_LOG = float(np.mean(np.log(np.arange(1, 31, dtype=np.float64))))
F32 = jnp.float32

N = 10000
E = 160000
D = 256
NW = 32
R = 320
NP = NW * R
NSR = 5
SR = R // NSR
NSB = NW * NSR
K = 64
CH = 1600
FL = 2048
ECAP = E + FL
CH2 = 512
FL2 = 1024
SCAP = E + 2 * FL2

_mesh = plsc.VectorSubcoreMesh(core_axis_name="c", subcore_axis_name="s",
                               num_cores=2, num_subcores=16)
_scparams = pltpu.CompilerParams(needs_layout_passes=False,
                                 use_tc_tiling_on_sc=False)


def _wid():
    return lax.axis_index("s") * 2 + lax.axis_index("c")


def _bin_body(dst_hbm, src_hbm,
              e1_hbm, se_hbm, counts_hbm, cnt_hbm,
              dbuf, st_e, cacc, cvec, ebuf2, dv2, st2, sem):
    w = _wid()
    lo = w * R
    iota = lax.iota(jnp.int32, 16)
    ones = jnp.ones((16,), F32)
    zeros = jnp.zeros((16,), F32)

    def initc(t, _):
        cacc[pl.ds(t * 16, 16)] = zeros
        return 0
    lax.fori_loop(0, R // 16, initc, 0)

    def chunk(ci, carry):
        pltpu.sync_copy(dst_hbm.at[pl.ds(ci * CH, CH)], dbuf)

        def vec(vi, c2):
            wp, off = c2
            d16 = dbuf[pl.ds(vi * 16, 16)]
            eid = ci * CH + vi * 16 + iota
            m = (d16 >= lo) & (d16 < lo + R)
            ld = jnp.clip(d16 - lo, 0, R - 1)
            plsc.addupdate_scatter(cacc, [ld], ones, mask=m)
            plsc.store_compressed(st_e.at[pl.ds(wp, 16)], eid, mask=m)
            npop = plsc.all_reduce_population_count(m)
            if npop.ndim:
                npop = npop[0]
            wp2 = wp + npop
            fl = wp2 >= FL

            @pl.when(fl)
            def _flush():
                pltpu.sync_copy(
                    st_e.at[pl.ds(0, FL)],
                    e1_hbm.at[pl.ds(pl.multiple_of(w * ECAP + off, 8), FL)])
                st_e[pl.ds(0, 16)] = st_e[pl.ds(FL, 16)]

            wp3 = jnp.where(fl, wp2 - FL, wp2)
            off2 = jnp.where(fl, off + FL, off)
            return wp3, off2

        return lax.fori_loop(0, CH // 16, vec, carry)

    wp, off = lax.fori_loop(0, E // CH, chunk,
                            (jnp.int32(0), jnp.int32(0)))
    pltpu.sync_copy(st_e.at[pl.ds(0, FL)],
                    e1_hbm.at[pl.ds(pl.multiple_of(w * ECAP + off, 8), FL)])
    cw = off + wp
    pltpu.sync_copy(cacc, cnt_hbm.at[pl.ds(pl.multiple_of(lo, 8), R)])

    nch = lax.div(cw + (CH2 - 1), jnp.int32(CH2))

    def chunk2(ci, carry):
        cbase = ci * CH2
        pltpu.sync_copy(
            e1_hbm.at[pl.ds(pl.multiple_of(w * ECAP + cbase, 8), CH2)],
            ebuf2)
        for t in range(CH2 // 16):
            mkv = (cbase + t * 16 + iota) < cw
            ebuf2[pl.ds(t * 16, 16)] = jnp.where(
                mkv, ebuf2[pl.ds(t * 16, 16)], 0)
        for g in range(CH2 // 128):
            pltpu.async_copy(dst_hbm.at[ebuf2.at[pl.ds(g * 128, 128)]],
                             dv2.at[pl.ds(g * 128, 128)], sem).wait()

        def vec2(vi, c2):
            wps = list(c2)
            valid = (cbase + vi * 16 + iota) < cw
            e16 = ebuf2[pl.ds(vi * 16, 16)]
            d16 = dv2[pl.ds(vi * 16, 16)]
            for s in range(NSR):
                slo = lo + s * SR
                m = valid & (d16 >= slo) & (d16 < slo + SR)
                plsc.store_compressed(st2.at[s, pl.ds(wps[s], 16)], e16,
                                      mask=m)
                npop = plsc.all_reduce_population_count(m)
                if npop.ndim:
                    npop = npop[0]
                wps[s] = wps[s] + npop
            return tuple(wps)

        wps = list(lax.fori_loop(0, CH2 // 16, vec2, tuple(carry[:NSR])))
        offs = list(carry[NSR:])
        for s in range(NSR):
            fl = wps[s] >= FL2

            @pl.when(fl)
            def _flush(s=s, off=offs[s]):
                pltpu.sync_copy(
                    st2.at[s, pl.ds(0, FL2)],
                    se_hbm.at[pl.ds(
                        pl.multiple_of((w * NSR + s) * SCAP + off, 8), FL2)])
                for t in range(CH2 // 16):
                    st2[s, pl.ds(t * 16, 16)] = \
                        st2[s, pl.ds(FL2 + t * 16, 16)]

            wps[s] = jnp.where(fl, wps[s] - FL2, wps[s])
            offs[s] = jnp.where(fl, offs[s] + FL2, offs[s])
        return tuple(wps) + tuple(offs)

    z = jnp.int32(0)
    carry = lax.fori_loop(0, nch, chunk2, (z,) * NSR + (z,) * NSR)
    for s in range(NSR):
        wps, offs = carry[s], carry[NSR + s]
        pltpu.sync_copy(
            st2.at[s, pl.ds(0, FL2)],
            se_hbm.at[pl.ds(
                pl.multiple_of((w * NSR + s) * SCAP + offs, 8), FL2)])

        @pl.when(wps > FL2)
        def _flush2(s=s, off2=offs + FL2):
            pltpu.sync_copy(
                st2.at[s, pl.ds(FL2, CH2)],
                se_hbm.at[pl.ds(
                    pl.multiple_of((w * NSR + s) * SCAP + off2, 8), CH2)])

        cvec[...] = jnp.full((16,), offs + wps, jnp.int32)
        pltpu.sync_copy(
            cvec.at[pl.ds(0, 8)],
            counts_hbm.at[pl.ds(pl.multiple_of((w * NSR + s) * 8, 8), 8)])


def _binning(dst, src):
    f = pl.kernel(
        _bin_body,
        compiler_params=_scparams,
        out_type=(
            jax.ShapeDtypeStruct((NW * ECAP,), jnp.int32),
            jax.ShapeDtypeStruct((NSB * SCAP,), jnp.int32),
            jax.ShapeDtypeStruct((NSB * 8,), jnp.int32),
            jax.ShapeDtypeStruct((NP,), F32),
        ),
        mesh=_mesh,
        scratch_types=[
            pltpu.VMEM((CH,), jnp.int32),
            pltpu.VMEM((FL + 16,), jnp.int32),
            pltpu.VMEM((R,), F32),
            pltpu.VMEM((16,), jnp.int32),
            pltpu.VMEM((CH2,), jnp.int32),
            pltpu.VMEM((CH2,), jnp.int32),
            pltpu.VMEM((NSR, FL2 + CH2 + 16), jnp.int32),
            pltpu.SemaphoreType.DMA,
        ],
    )
    return f(dst, src)


def _scmain_body(se_hbm, counts_hbm, dst_hbm, src_hbm, xd_hbm, xs_hbm,
                 ee_hbm,
                 sum_hbm, sq_hbm, mx_hbm, mn_hbm,
                 cbuf, xdloc, rxs, ree, ev, dval, sval,
                 acc_s, acc_q, acc_mx, acc_mn, sem0, sem1, sem2, sem3):
    w = _wid()
    iota = lax.iota(jnp.int32, 16)
    zeros = jnp.zeros((16,), F32)
    neg = jnp.full((16,), -3.0e38, F32)
    pos = jnp.full((16,), 3.0e38, F32)

    pltpu.sync_copy(counts_hbm, cbuf.at[pl.ds(0, NSB * 8)])

    for s in range(NSR):
        lo_s = w * R + s * SR
        sb = w * NSR + s
        sc = cbuf[pl.ds(sb * 8, 16)][0]
        nblk = lax.div(sc + (K - 1), jnp.int32(K))

        def initrow(r, _):
            for v in range(D // 16):
                cs = pl.ds(v * 16, 16)
                acc_s[r, cs] = zeros
                acc_q[r, cs] = zeros
                acc_mx[r, cs] = neg
                acc_mn[r, cs] = pos
            return 0
        lax.fori_loop(0, SR, initrow, 0)

        pltpu.sync_copy(
            xd_hbm.at[pl.ds(pl.multiple_of(lo_s, 8), SR)], xdloc)

        def blk(b, _):
            base = b * K
            pltpu.sync_copy(
                se_hbm.at[pl.ds(pl.multiple_of(sb * SCAP + base, 8), K)],
                ev)
            nval = jnp.minimum(sc - base, K)
            for t in range(K // 16):
                mk = (t * 16 + iota) < nval
                ev[pl.ds(t * 16, 16)] = jnp.where(
                    mk, ev[pl.ds(t * 16, 16)], 0)
            c0 = pltpu.async_copy(dst_hbm.at[ev], dval.at[pl.ds(0, K)],
                                  sem0)
            c1 = pltpu.async_copy(src_hbm.at[ev], sval, sem1)
            c2 = pltpu.async_copy(ee_hbm.at[ev], ree, sem2)
            c1.wait()
            c3 = pltpu.async_copy(xs_hbm.at[sval], rxs, sem3)
            c0.wait()
            c2.wait()
            c3.wait()

            def edge(j, _):
                r = dval[pl.ds(j, 16)][0] - lo_s
                for v in range(D // 16):
                    cs = pl.ds(v * 16, 16)
                    mv = xdloc[r, cs] + rxs[j, cs] + ree[j, cs]
                    plsc.addupdate(acc_s.at[r, cs], mv)
                    plsc.addupdate(acc_q.at[r, cs], mv * mv)
                    acc_mx[r, cs] = jnp.maximum(acc_mx[r, cs], mv)
                    acc_mn[r, cs] = jnp.minimum(acc_mn[r, cs], mv)
                return 0
            lax.fori_loop(0, nval, edge, 0)
            return 0

        lax.fori_loop(0, nblk, blk, 0)

        pltpu.sync_copy(acc_s,
                        sum_hbm.at[pl.ds(pl.multiple_of(lo_s, 8), SR)])
        pltpu.sync_copy(acc_q,
                        sq_hbm.at[pl.ds(pl.multiple_of(lo_s, 8), SR)])
        pltpu.sync_copy(acc_mx,
                        mx_hbm.at[pl.ds(pl.multiple_of(lo_s, 8), SR)])
        pltpu.sync_copy(acc_mn,
                        mn_hbm.at[pl.ds(pl.multiple_of(lo_s, 8), SR)])


def _scmain(se, counts, dst, src, xd, xs, ee):
    f = pl.kernel(
        _scmain_body,
        compiler_params=_scparams,
        out_type=(
            jax.ShapeDtypeStruct((NP, D), F32),
            jax.ShapeDtypeStruct((NP, D), F32),
            jax.ShapeDtypeStruct((NP, D), F32),
            jax.ShapeDtypeStruct((NP, D), F32),
        ),
        mesh=_mesh,
        scratch_types=[
            pltpu.VMEM((NSB * 8 + 16,), jnp.int32),
            pltpu.VMEM((SR, D), F32),
            pltpu.VMEM((K, D), F32),
            pltpu.VMEM((K, D), F32),
            pltpu.VMEM((K,), jnp.int32),
            pltpu.VMEM((K + 16,), jnp.int32),
            pltpu.VMEM((K,), jnp.int32),
            pltpu.VMEM((SR, D), F32),
            pltpu.VMEM((SR, D), F32),
            pltpu.VMEM((SR, D), F32),
            pltpu.VMEM((SR, D), F32),
            pltpu.SemaphoreType.DMA,
            pltpu.SemaphoreType.DMA,
            pltpu.SemaphoreType.DMA,
            pltpu.SemaphoreType.DMA,
        ],
    )
    return f(se, counts, dst, src, xd, xs, ee)


def _wprep_body(We_ref, Wpe_ref, be_ref, bpre_ref, Wc_ref, cb_ref):
    Wc_ref[...] = jnp.dot(We_ref[...], Wpe_ref[...],
                          preferred_element_type=F32, precision=lax.Precision.HIGHEST)
    cb_ref[...] = jnp.dot(be_ref[...], Wpe_ref[...],
                          preferred_element_type=F32, precision=lax.Precision.HIGHEST) + bpre_ref[...]


def _wprep(We, Wpe, be, bpre):
    de, d = We.shape
    return pl.pallas_call(
        _wprep_body,
        out_shape=(jax.ShapeDtypeStruct((de, d), F32),
                   jax.ShapeDtypeStruct((1, d), F32)),
    )(We, Wpe, be.reshape(1, -1), bpre.reshape(1, -1))


def _xprep_body(x_ref, Wd_ref, Ws_ref, xd_ref, xs_ref):
    xb = x_ref[...]
    xd_ref[...] = jnp.dot(xb, Wd_ref[...], preferred_element_type=F32, precision=lax.Precision.HIGHEST)
    xs_ref[...] = jnp.dot(xb, Ws_ref[...], preferred_element_type=F32, precision=lax.Precision.HIGHEST)


def _xprep(x, Wd, Ws):
    d = x.shape[1]
    nb = NP // R
    return pl.pallas_call(
        _xprep_body,
        grid=(nb,),
        in_specs=[
            pl.BlockSpec((R, d), lambda i: (i, 0)),
            pl.BlockSpec((d, D), lambda i: (0, 0)),
            pl.BlockSpec((d, D), lambda i: (0, 0)),
        ],
        out_specs=(
            pl.BlockSpec((R, D), lambda i: (i, 0)),
            pl.BlockSpec((R, D), lambda i: (i, 0)),
        ),
        out_shape=(jax.ShapeDtypeStruct((NP, D), F32),
                   jax.ShapeDtypeStruct((NP, D), F32)),
    )(x, Wd, Ws)


def _eeprep_body(ea_ref, Wc_ref, cb_ref, ee_ref):
    ee_ref[...] = jnp.dot(ea_ref[...], Wc_ref[...],
                          preferred_element_type=F32, precision=lax.Precision.HIGHEST) + cb_ref[...]


def _eeprep(ea, Wc, cb):
    de = ea.shape[1]
    eb = 640
    nb = E // eb
    return pl.pallas_call(
        _eeprep_body,
        grid=(nb,),
        in_specs=[
            pl.BlockSpec((eb, de), lambda i: (i, 0)),
            pl.BlockSpec((de, D), lambda i: (0, 0)),
            pl.BlockSpec((1, D), lambda i: (0, 0)),
        ],
        out_specs=pl.BlockSpec((eb, D), lambda i: (i, 0)),
        out_shape=jax.ShapeDtypeStruct((E, D), F32),
    )(ea, Wc, cb)


def _post_body(x_ref, cnt_ref, s_ref, q_ref, mx_ref, mn_ref,
               Wa_ref, Wb_ref, Wc_ref, bpost_ref, Wlin_ref, blin_ref,
               h_ref, st_ref):
    i = pl.program_id(0)
    cnt = cnt_ref[...]
    cntc = jnp.maximum(cnt, 1.0)
    posm = cnt > 0.0
    mean = s_ref[...] / cntc
    msq = q_ref[...] / cntc
    std = jnp.sqrt(jnp.maximum(msq - mean * mean, 0.0) + 1e-5)
    mx = jnp.where(posm, mx_ref[...], 0.0)
    mn = jnp.where(posm, mn_ref[...], 0.0)
    agg = jnp.concatenate([mean, mx, mn, std], axis=1)
    lc = jnp.log(cntc + 1.0)
    amp = lc / AVG_LOG
    att = AVG_LOG / lc
    cat = jnp.concatenate([x_ref[...], agg], axis=1)
    P = jnp.dot(cat, Wa_ref[...], preferred_element_type=F32, precision=lax.Precision.HIGHEST) + bpost_ref[...]
    P = P + amp * jnp.dot(agg, Wb_ref[...], preferred_element_type=F32, precision=lax.Precision.HIGHEST)
    P = P + att * jnp.dot(agg, Wc_ref[...], preferred_element_type=F32, precision=lax.Precision.HIGHEST)
    H = jnp.dot(P, Wlin_ref[...], preferred_element_type=F32, precision=lax.Precision.HIGHEST) + blin_ref[...]
    h_ref[...] = H
    cs = jnp.sum(H, axis=0, keepdims=True)
    st = jnp.concatenate(
        [cs, jnp.zeros((7, cs.shape[1]), F32)], axis=0)

    @pl.when(i == 0)
    def _():
        st_ref[...] = st

    @pl.when(i > 0)
    def _():
        st_ref[...] = st_ref[...] + st


def _post(x, cnt, s4, q4, mx4, mn4, Wa, Wb, Wc, bpost, Wlin, blin):
    d = x.shape[1]
    h = Wlin.shape[1]
    rb = 1000
    nb = N // rb
    return pl.pallas_call(
        _post_body,
        grid=(nb,),
        in_specs=[
            pl.BlockSpec((rb, d), lambda i: (i, 0)),
            pl.BlockSpec((rb, 1), lambda i: (i, 0)),
            pl.BlockSpec((rb, D), lambda i: (i, 0)),
            pl.BlockSpec((rb, D), lambda i: (i, 0)),
            pl.BlockSpec((rb, D), lambda i: (i, 0)),
            pl.BlockSpec((rb, D), lambda i: (i, 0)),
            pl.BlockSpec(Wa.shape, lambda i: (0, 0)),
            pl.BlockSpec(Wb.shape, lambda i: (0, 0)),
            pl.BlockSpec(Wc.shape, lambda i: (0, 0)),
            pl.BlockSpec((1, h), lambda i: (0, 0)),
            pl.BlockSpec(Wlin.shape, lambda i: (0, 0)),
            pl.BlockSpec((1, h), lambda i: (0, 0)),
        ],
        out_specs=(
            pl.BlockSpec((rb, h), lambda i: (i, 0)),
            pl.BlockSpec((8, h), lambda i: (0, 0)),
        ),
        out_shape=(jax.ShapeDtypeStruct((N, h), F32),
                   jax.ShapeDtypeStruct((8, h), F32)),
    )(x, cnt, s4, q4, mx4, mn4, Wa, Wb, Wc, bpost, Wlin, blin)


def _var_body(h_ref, st_ref, v_ref):
    i = pl.program_id(0)
    mu = st_ref[0:1] / N
    dd = h_ref[...] - mu
    vs = jnp.sum(dd * dd, axis=0, keepdims=True)
    vv = jnp.concatenate([vs, jnp.zeros((7, vs.shape[1]), F32)], axis=0)

    @pl.when(i == 0)
    def _():
        v_ref[...] = vv

    @pl.when(i > 0)
    def _():
        v_ref[...] = v_ref[...] + vv


def _varpass(H, stats):
    h = H.shape[1]
    rb = 1000
    return pl.pallas_call(
        _var_body,
        grid=(N // rb,),
        in_specs=[
            pl.BlockSpec((rb, h), lambda i: (i, 0)),
            pl.BlockSpec((8, h), lambda i: (0, 0)),
        ],
        out_specs=pl.BlockSpec((8, h), lambda i: (0, 0)),
        out_shape=jax.ShapeDtypeStruct((8, h), F32),
    )(H, stats)


def _bn_body(h_ref, st_ref, vr_ref, g_ref, b_ref, o_ref):
    mu = st_ref[0:1] / N
    var = vr_ref[0:1] / N
    y = g_ref[...] * (h_ref[...] - mu) / jnp.sqrt(var + 1e-5) + b_ref[...]
    o_ref[...] = jnp.where(y > 0, y, jnp.exp(y) - 1.0)


def _bn_apply(H, stats, g, b):
    h = H.shape[1]
    rb = 1000
    vr = _varpass(H, stats)
    return pl.pallas_call(
        _bn_body,
        grid=(N // rb,),
        in_specs=[
            pl.BlockSpec((rb, h), lambda i: (i, 0)),
            pl.BlockSpec((8, h), lambda i: (0, 0)),
            pl.BlockSpec((8, h), lambda i: (0, 0)),
            pl.BlockSpec((1, h), lambda i: (0, 0)),
            pl.BlockSpec((1, h), lambda i: (0, 0)),
        ],
        out_specs=pl.BlockSpec((rb, h), lambda i: (i, 0)),
        out_shape=jax.ShapeDtypeStruct((N, h), F32),
    )(H, stats, vr, g.reshape(1, -1), b.reshape(1, -1))


def _bn_head_body(h_ref, st_ref, vr_ref, g_ref, b_ref, Wo_ref, bo_ref, o_ref):
    mu = st_ref[0:1] / N
    var = vr_ref[0:1] / N
    y = g_ref[...] * (h_ref[...] - mu) / jnp.sqrt(var + 1e-5) + b_ref[...]
    y = jnp.where(y > 0, y, jnp.exp(y) - 1.0)
    o_ref[...] = jnp.dot(y, Wo_ref[...], preferred_element_type=F32, precision=lax.Precision.HIGHEST) \
        + bo_ref[...]


def _bn_head(H, stats, g, b, Wout, bout):
    h = H.shape[1]
    rb = 1000
    vr = _varpass(H, stats)
    return pl.pallas_call(
        _bn_head_body,
        grid=(N // rb,),
        in_specs=[
            pl.BlockSpec((rb, h), lambda i: (i, 0)),
            pl.BlockSpec((8, h), lambda i: (0, 0)),
            pl.BlockSpec((8, h), lambda i: (0, 0)),
            pl.BlockSpec((1, h), lambda i: (0, 0)),
            pl.BlockSpec((1, h), lambda i: (0, 0)),
            pl.BlockSpec((h, 1), lambda i: (0, 0)),
            pl.BlockSpec((1, 1), lambda i: (0, 0)),
        ],
        out_specs=pl.BlockSpec((rb, 1), lambda i: (i, 0)),
        out_shape=jax.ShapeDtypeStruct((N, 1), F32),
    )(H, stats, vr, g.reshape(1, -1), b.reshape(1, -1), Wout,
      bout.reshape(1, 1))


def _layer(x, ea, bins, dst, src, We, be, Wpre, bpre, Wpost, bpost,
           Wlin, blin):
    se, counts, cnt = bins
    d = x.shape[1]
    Wd, Ws, Wpe = Wpre[:d], Wpre[d:2 * d], Wpre[2 * d:]
    Wc, cb = _wprep(We, Wpe, be, bpre)
    xd, xs = _xprep(x, Wd, Ws)
    ee = _eeprep(ea, Wc, cb)
    s4, q4, mx4, mn4 = _scmain(se, counts, dst, src, xd, xs, ee)
    cnt2 = cnt[:N].reshape(N, 1)
    Wa = Wpost[:d + 4 * d]
    Wb = Wpost[d + 4 * d:d + 8 * d]
    Wcg = Wpost[d + 8 * d:d + 12 * d]
    return _post(x, cnt2, s4, q4, mx4, mn4, Wa, Wb, Wcg,
                 bpost.reshape(1, -1), Wlin, blin.reshape(1, -1))


def kernel(x, edge_index, edge_attr, We1, be1, Wpre1, bpre1, Wpost1, bpost1,
           Wlin1, blin1, g1, bb1, We2, be2, Wpre2, bpre2, Wpost2, bpost2,
           Wlin2, blin2, g2, bb2, Wout, bout):
    src = edge_index[0]
    dst = edge_index[1]
    e1, se, counts, cnt = _binning(dst, src)
    bins = (se, counts, cnt)
    H1, st1 = _layer(x, edge_attr, bins, dst, src, We1, be1, Wpre1, bpre1,
                     Wpost1, bpost1, Wlin1, blin1)
    h1 = _bn_apply(H1, st1, g1, bb1)
    H2, st2 = _layer(h1, edge_attr, bins, dst, src, We2, be2, Wpre2, bpre2,
                     Wpost2, bpost2, Wlin2, blin2)
    return _bn_head(H2, st2, g2, bb2, Wout, bout)


# speedup vs baseline: 2.3690x; 1.0530x over previous
import numpy as np
import jax
import jax.numpy as jnp
from jax import lax
from jax.experimental import pallas as pl
from jax.experimental.pallas import tpu as pltpu
from jax.experimental.pallas import tpu_sc as plsc

AVG_LOG = float(np.mean(np.log(np.arange(1, 31, dtype=np.float64))))
F32 = jnp.float32

N = 10000
E = 160000
D = 256
NW = 32
R = 320
NP = NW * R
NSR = 8
SR = R // NSR
NSB = NW * NSR
K = 64
CH = 1600
FL = 2048
ECAP = E + FL
CH2 = 512
FL2 = 1024
SCAP = E + 2 * FL2

_mesh = plsc.VectorSubcoreMesh(core_axis_name="c", subcore_axis_name="s",
                               num_cores=2, num_subcores=16)
_scparams = pltpu.CompilerParams(needs_layout_passes=False,
                                 use_tc_tiling_on_sc=False)


def _wid():
    return lax.axis_index("s") * 2 + lax.axis_index("c")


def _bin_body(dst_hbm, src_hbm,
              e1_hbm, se_hbm, sd_hbm, ss_hbm, counts_hbm, cnt_hbm,
              dbuf, st_e, cacc, cvec, ebuf2, dv2, sv2, st2e, st2d, st2s,
              sem, sem2):
    w = _wid()
    lo = w * R
    iota = lax.iota(jnp.int32, 16)
    ones = jnp.ones((16,), F32)
    zeros = jnp.zeros((16,), F32)

    def initc(t, _):
        cacc[pl.ds(t * 16, 16)] = zeros
        return 0
    lax.fori_loop(0, R // 16, initc, 0)

    def chunk(ci, carry):
        pltpu.sync_copy(dst_hbm.at[pl.ds(ci * CH, CH)], dbuf)

        def vec(vi, c2):
            wp, off = c2
            d16 = dbuf[pl.ds(vi * 16, 16)]
            eid = ci * CH + vi * 16 + iota
            m = (d16 >= lo) & (d16 < lo + R)
            ld = jnp.clip(d16 - lo, 0, R - 1)
            plsc.addupdate_scatter(cacc, [ld], ones, mask=m)
            plsc.store_compressed(st_e.at[pl.ds(wp, 16)], eid, mask=m)
            npop = plsc.all_reduce_population_count(m)
            if npop.ndim:
                npop = npop[0]
            wp2 = wp + npop
            fl = wp2 >= FL

            @pl.when(fl)
            def _flush():
                pltpu.sync_copy(
                    st_e.at[pl.ds(0, FL)],
                    e1_hbm.at[pl.ds(pl.multiple_of(w * ECAP + off, 8), FL)])
                st_e[pl.ds(0, 16)] = st_e[pl.ds(FL, 16)]

            wp3 = jnp.where(fl, wp2 - FL, wp2)
            off2 = jnp.where(fl, off + FL, off)
            return wp3, off2

        return lax.fori_loop(0, CH // 16, vec, carry)

    wp, off = lax.fori_loop(0, E // CH, chunk,
                            (jnp.int32(0), jnp.int32(0)))
    pltpu.sync_copy(st_e.at[pl.ds(0, FL)],
                    e1_hbm.at[pl.ds(pl.multiple_of(w * ECAP + off, 8), FL)])
    cw = off + wp
    pltpu.sync_copy(cacc, cnt_hbm.at[pl.ds(pl.multiple_of(lo, 8), R)])

    nch = lax.div(cw + (CH2 - 1), jnp.int32(CH2))

    def chunk2(ci, carry):
        cbase = ci * CH2
        pltpu.sync_copy(
            e1_hbm.at[pl.ds(pl.multiple_of(w * ECAP + cbase, 8), CH2)],
            ebuf2)
        for t in range(CH2 // 16):
            mkv = (cbase + t * 16 + iota) < cw
            ebuf2[pl.ds(t * 16, 16)] = jnp.where(
                mkv, ebuf2[pl.ds(t * 16, 16)], 0)
        for g in range(CH2 // 128):
            c0 = pltpu.async_copy(dst_hbm.at[ebuf2.at[pl.ds(g * 128, 128)]],
                                  dv2.at[pl.ds(g * 128, 128)], sem)
            c1 = pltpu.async_copy(src_hbm.at[ebuf2.at[pl.ds(g * 128, 128)]],
                                  sv2.at[pl.ds(g * 128, 128)], sem2)
            c0.wait()
            c1.wait()

        def vec2(vi, c2):
            wps = list(c2)
            valid = (cbase + vi * 16 + iota) < cw
            e16 = ebuf2[pl.ds(vi * 16, 16)]
            d16 = dv2[pl.ds(vi * 16, 16)]
            s16 = sv2[pl.ds(vi * 16, 16)]
            for s in range(NSR):
                slo = lo + s * SR
                m = valid & (d16 >= slo) & (d16 < slo + SR)
                plsc.store_compressed(st2e.at[s, pl.ds(wps[s], 16)], e16,
                                      mask=m)
                plsc.store_compressed(st2d.at[s, pl.ds(wps[s], 16)], d16,
                                      mask=m)
                plsc.store_compressed(st2s.at[s, pl.ds(wps[s], 16)], s16,
                                      mask=m)
                npop = plsc.all_reduce_population_count(m)
                if npop.ndim:
                    npop = npop[0]
                wps[s] = wps[s] + npop
            return tuple(wps)

        wps = list(lax.fori_loop(0, CH2 // 16, vec2, tuple(carry[:NSR])))
        offs = list(carry[NSR:])
        for s in range(NSR):
            fl = wps[s] >= FL2

            @pl.when(fl)
            def _flush(s=s, off=offs[s]):
                for st2, bh in ((st2e, se_hbm), (st2d, sd_hbm),
                                (st2s, ss_hbm)):
                    pltpu.sync_copy(
                        st2.at[s, pl.ds(0, FL2)],
                        bh.at[pl.ds(
                            pl.multiple_of((w * NSR + s) * SCAP + off, 8),
                            FL2)])
                    for t in range(CH2 // 16):
                        st2[s, pl.ds(t * 16, 16)] = \
                            st2[s, pl.ds(FL2 + t * 16, 16)]

            wps[s] = jnp.where(fl, wps[s] - FL2, wps[s])
            offs[s] = jnp.where(fl, offs[s] + FL2, offs[s])
        return tuple(wps) + tuple(offs)

    z = jnp.int32(0)
    carry = lax.fori_loop(0, nch, chunk2, (z,) * NSR + (z,) * NSR)
    for s in range(NSR):
        wps, offs = carry[s], carry[NSR + s]
        for st2, bh in ((st2e, se_hbm), (st2d, sd_hbm), (st2s, ss_hbm)):
            pltpu.sync_copy(
                st2.at[s, pl.ds(0, FL2)],
                bh.at[pl.ds(
                    pl.multiple_of((w * NSR + s) * SCAP + offs, 8), FL2)])

            @pl.when(wps > FL2)
            def _flush2(st2=st2, bh=bh, s=s, off2=offs + FL2):
                pltpu.sync_copy(
                    st2.at[s, pl.ds(FL2, CH2)],
                    bh.at[pl.ds(
                        pl.multiple_of((w * NSR + s) * SCAP + off2, 8),
                        CH2)])

        cvec[...] = jnp.full((16,), offs + wps, jnp.int32)
        pltpu.sync_copy(
            cvec.at[pl.ds(0, 8)],
            counts_hbm.at[pl.ds(pl.multiple_of((w * NSR + s) * 8, 8), 8)])


def _binning(dst, src):
    f = pl.kernel(
        _bin_body,
        compiler_params=_scparams,
        out_type=(
            jax.ShapeDtypeStruct((NW * ECAP,), jnp.int32),
            jax.ShapeDtypeStruct((NSB * SCAP,), jnp.int32),
            jax.ShapeDtypeStruct((NSB * SCAP,), jnp.int32),
            jax.ShapeDtypeStruct((NSB * SCAP,), jnp.int32),
            jax.ShapeDtypeStruct((NSB * 8,), jnp.int32),
            jax.ShapeDtypeStruct((NP,), F32),
        ),
        mesh=_mesh,
        scratch_types=[
            pltpu.VMEM((CH,), jnp.int32),
            pltpu.VMEM((FL + 16,), jnp.int32),
            pltpu.VMEM((R,), F32),
            pltpu.VMEM((16,), jnp.int32),
            pltpu.VMEM((CH2,), jnp.int32),
            pltpu.VMEM((CH2,), jnp.int32),
            pltpu.VMEM((CH2,), jnp.int32),
            pltpu.VMEM((NSR, FL2 + CH2 + 16), jnp.int32),
            pltpu.VMEM((NSR, FL2 + CH2 + 16), jnp.int32),
            pltpu.VMEM((NSR, FL2 + CH2 + 16), jnp.int32),
            pltpu.SemaphoreType.DMA,
            pltpu.SemaphoreType.DMA,
        ],
    )
    return f(dst, src)


def _scmain_body(se_hbm, sd_hbm, ss_hbm, counts_hbm, xd_hbm, xs_hbm,
                 ee_hbm,
                 sum_hbm, sq_hbm, mx_hbm, mn_hbm,
                 cbuf, xdloc,
                 rxsA, reeA, evA, dvA, svA,
                 rxsB, reeB, evB, dvB, svB,
                 acc_s, acc_q, acc_mx, acc_mn,
                 semeA, semxA, semeB, semxB):
    w = _wid()
    iota = lax.iota(jnp.int32, 16)
    zeros = jnp.zeros((16,), F32)
    neg = jnp.full((16,), -3.0e38, F32)
    pos = jnp.full((16,), 3.0e38, F32)

    pltpu.sync_copy(counts_hbm, cbuf.at[pl.ds(0, NSB * 8)])

    for s in range(NSR):
        lo_s = w * R + s * SR
        sb = w * NSR + s
        sc = cbuf[pl.ds(sb * 8, 16)][0]
        nblk = lax.div(sc + (K - 1), jnp.int32(K))
        npair = lax.div(nblk + 1, jnp.int32(2))

        def initrow(r, _):
            for v in range(D // 16):
                cs = pl.ds(v * 16, 16)
                acc_s[r, cs] = zeros
                acc_q[r, cs] = zeros
                acc_mx[r, cs] = neg
                acc_mn[r, cs] = pos
            return 0
        lax.fori_loop(0, SR, initrow, 0)

        pltpu.sync_copy(
            xd_hbm.at[pl.ds(pl.multiple_of(lo_s, 8), SR)], xdloc)

        def issue(b, ev, dv, sv, rxs, ree, seme, semx):
            base = b * K
            pltpu.sync_copy(
                se_hbm.at[pl.ds(pl.multiple_of(sb * SCAP + base, 8), K)],
                ev)
            pltpu.sync_copy(
                sd_hbm.at[pl.ds(pl.multiple_of(sb * SCAP + base, 8), K)],
                dv.at[pl.ds(0, K)])
            pltpu.sync_copy(
                ss_hbm.at[pl.ds(pl.multiple_of(sb * SCAP + base, 8), K)],
                sv)
            nval = jnp.minimum(sc - base, K)
            for t in range(K // 16):
                mk = (t * 16 + iota) < nval
                ev[pl.ds(t * 16, 16)] = jnp.where(
                    mk, ev[pl.ds(t * 16, 16)], 0)
                sv[pl.ds(t * 16, 16)] = jnp.where(
                    mk, sv[pl.ds(t * 16, 16)], 0)
            pltpu.async_copy(ee_hbm.at[ev], ree, seme)
            pltpu.async_copy(xs_hbm.at[sv], rxs, semx)

        def consume(b, ev, dv, sv, rxs, ree, seme, semx):
            pltpu.make_async_copy(ee_hbm.at[ev], ree, seme).wait()
            pltpu.make_async_copy(xs_hbm.at[sv], rxs, semx).wait()
            nval = jnp.minimum(sc - b * K, K)

            def edge(j, _):
                r = dv[pl.ds(j, 16)][0] - lo_s
                for v in range(D // 16):
                    cs = pl.ds(v * 16, 16)
                    mv = xdloc[r, cs] + rxs[j, cs] + ree[j, cs]
                    plsc.addupdate(acc_s.at[r, cs], mv)
                    plsc.addupdate(acc_q.at[r, cs], mv * mv)
                    acc_mx[r, cs] = jnp.maximum(acc_mx[r, cs], mv)
                    acc_mn[r, cs] = jnp.minimum(acc_mn[r, cs], mv)
                return 0
            lax.fori_loop(0, nval, edge, 0)

        bufA = (evA, dvA, svA, rxsA, reeA, semeA, semxA)
        bufB = (evB, dvB, svB, rxsB, reeB, semeB, semxB)

        @pl.when(nblk > 0)
        def _():
            issue(jnp.int32(0), bufA[0], bufA[1], bufA[2], bufA[3],
                  bufA[4], bufA[5], bufA[6])

        def pair(p, _):
            b0 = 2 * p
            b1 = 2 * p + 1

            @pl.when(b1 < nblk)
            def _():
                issue(b1, bufB[0], bufB[1], bufB[2], bufB[3], bufB[4],
                      bufB[5], bufB[6])

            consume(b0, bufA[0], bufA[1], bufA[2], bufA[3], bufA[4],
                    bufA[5], bufA[6])

            @pl.when(b1 + 1 < nblk)
            def _():
                issue(b1 + 1, bufA[0], bufA[1], bufA[2], bufA[3], bufA[4],
                      bufA[5], bufA[6])

            @pl.when(b1 < nblk)
            def _():
                consume(b1, bufB[0], bufB[1], bufB[2], bufB[3], bufB[4],
                        bufB[5], bufB[6])
            return 0

        lax.fori_loop(0, npair, pair, 0)

        pltpu.sync_copy(acc_s,
                        sum_hbm.at[pl.ds(pl.multiple_of(lo_s, 8), SR)])
        pltpu.sync_copy(acc_q,
                        sq_hbm.at[pl.ds(pl.multiple_of(lo_s, 8), SR)])
        pltpu.sync_copy(acc_mx,
                        mx_hbm.at[pl.ds(pl.multiple_of(lo_s, 8), SR)])
        pltpu.sync_copy(acc_mn,
                        mn_hbm.at[pl.ds(pl.multiple_of(lo_s, 8), SR)])


def _scmain(se, sd, ss, counts, xd, xs, ee):
    f = pl.kernel(
        _scmain_body,
        compiler_params=_scparams,
        out_type=(
            jax.ShapeDtypeStruct((NP, D), F32),
            jax.ShapeDtypeStruct((NP, D), F32),
            jax.ShapeDtypeStruct((NP, D), F32),
            jax.ShapeDtypeStruct((NP, D), F32),
        ),
        mesh=_mesh,
        scratch_types=[
            pltpu.VMEM((NSB * 8 + 16,), jnp.int32),
            pltpu.VMEM((SR, D), F32),
            pltpu.VMEM((K, D), F32),
            pltpu.VMEM((K, D), F32),
            pltpu.VMEM((K,), jnp.int32),
            pltpu.VMEM((K + 16,), jnp.int32),
            pltpu.VMEM((K,), jnp.int32),
            pltpu.VMEM((K, D), F32),
            pltpu.VMEM((K, D), F32),
            pltpu.VMEM((K,), jnp.int32),
            pltpu.VMEM((K + 16,), jnp.int32),
            pltpu.VMEM((K,), jnp.int32),
            pltpu.VMEM((SR, D), F32),
            pltpu.VMEM((SR, D), F32),
            pltpu.VMEM((SR, D), F32),
            pltpu.VMEM((SR, D), F32),
            pltpu.SemaphoreType.DMA,
            pltpu.SemaphoreType.DMA,
            pltpu.SemaphoreType.DMA,
            pltpu.SemaphoreType.DMA,
        ],
    )
    return f(se, sd, ss, counts, xd, xs, ee)


def _wprep_body(We_ref, Wpe_ref, be_ref, bpre_ref, Wc_ref, cb_ref):
    Wc_ref[...] = jnp.dot(We_ref[...], Wpe_ref[...],
                          preferred_element_type=F32, precision=lax.Precision.HIGHEST)
    cb_ref[...] = jnp.dot(be_ref[...], Wpe_ref[...],
                          preferred_element_type=F32, precision=lax.Precision.HIGHEST) + bpre_ref[...]


def _wprep(We, Wpe, be, bpre):
    de, d = We.shape
    return pl.pallas_call(
        _wprep_body,
        out_shape=(jax.ShapeDtypeStruct((de, d), F32),
                   jax.ShapeDtypeStruct((1, d), F32)),
    )(We, Wpe, be.reshape(1, -1), bpre.reshape(1, -1))


def _xprep_body(x_ref, Wd_ref, Ws_ref, xd_ref, xs_ref):
    xb = x_ref[...]
    xd_ref[...] = jnp.dot(xb, Wd_ref[...], preferred_element_type=F32, precision=lax.Precision.HIGHEST)
    xs_ref[...] = jnp.dot(xb, Ws_ref[...], preferred_element_type=F32, precision=lax.Precision.HIGHEST)


def _xprep(x, Wd, Ws):
    d = x.shape[1]
    nb = NP // R
    return pl.pallas_call(
        _xprep_body,
        grid=(nb,),
        in_specs=[
            pl.BlockSpec((R, d), lambda i: (i, 0)),
            pl.BlockSpec((d, D), lambda i: (0, 0)),
            pl.BlockSpec((d, D), lambda i: (0, 0)),
        ],
        out_specs=(
            pl.BlockSpec((R, D), lambda i: (i, 0)),
            pl.BlockSpec((R, D), lambda i: (i, 0)),
        ),
        out_shape=(jax.ShapeDtypeStruct((NP, D), F32),
                   jax.ShapeDtypeStruct((NP, D), F32)),
    )(x, Wd, Ws)


def _eeprep_body(ea_ref, Wc_ref, cb_ref, ee_ref):
    ee_ref[...] = jnp.dot(ea_ref[...], Wc_ref[...],
                          preferred_element_type=F32, precision=lax.Precision.HIGHEST) + cb_ref[...]


def _eeprep(ea, Wc, cb):
    de = ea.shape[1]
    eb = 640
    nb = E // eb
    return pl.pallas_call(
        _eeprep_body,
        grid=(nb,),
        in_specs=[
            pl.BlockSpec((eb, de), lambda i: (i, 0)),
            pl.BlockSpec((de, D), lambda i: (0, 0)),
            pl.BlockSpec((1, D), lambda i: (0, 0)),
        ],
        out_specs=pl.BlockSpec((eb, D), lambda i: (i, 0)),
        out_shape=jax.ShapeDtypeStruct((E, D), F32),
    )(ea, Wc, cb)


def _post_body(x_ref, cnt_ref, s_ref, q_ref, mx_ref, mn_ref,
               Wa_ref, Wb_ref, Wc_ref, bpost_ref, Wlin_ref, blin_ref,
               h_ref, st_ref):
    i = pl.program_id(0)
    cnt = cnt_ref[...]
    cntc = jnp.maximum(cnt, 1.0)
    posm = cnt > 0.0
    mean = s_ref[...] / cntc
    msq = q_ref[...] / cntc
    std = jnp.sqrt(jnp.maximum(msq - mean * mean, 0.0) + 1e-5)
    mx = jnp.where(posm, mx_ref[...], 0.0)
    mn = jnp.where(posm, mn_ref[...], 0.0)
    agg = jnp.concatenate([mean, mx, mn, std], axis=1)
    lc = jnp.log(cntc + 1.0)
    amp = lc / AVG_LOG
    att = AVG_LOG / lc
    cat = jnp.concatenate([x_ref[...], agg], axis=1)
    P = jnp.dot(cat, Wa_ref[...], preferred_element_type=F32, precision=lax.Precision.HIGHEST) + bpost_ref[...]
    P = P + amp * jnp.dot(agg, Wb_ref[...], preferred_element_type=F32, precision=lax.Precision.HIGHEST)
    P = P + att * jnp.dot(agg, Wc_ref[...], preferred_element_type=F32, precision=lax.Precision.HIGHEST)
    H = jnp.dot(P, Wlin_ref[...], preferred_element_type=F32, precision=lax.Precision.HIGHEST) + blin_ref[...]
    h_ref[...] = H
    cs = jnp.sum(H, axis=0, keepdims=True)
    st = jnp.concatenate(
        [cs, jnp.zeros((7, cs.shape[1]), F32)], axis=0)

    @pl.when(i == 0)
    def _():
        st_ref[...] = st

    @pl.when(i > 0)
    def _():
        st_ref[...] = st_ref[...] + st


def _post(x, cnt, s4, q4, mx4, mn4, Wa, Wb, Wc, bpost, Wlin, blin):
    d = x.shape[1]
    h = Wlin.shape[1]
    rb = 1000
    nb = N // rb
    return pl.pallas_call(
        _post_body,
        grid=(nb,),
        in_specs=[
            pl.BlockSpec((rb, d), lambda i: (i, 0)),
            pl.BlockSpec((rb, 1), lambda i: (i, 0)),
            pl.BlockSpec((rb, D), lambda i: (i, 0)),
            pl.BlockSpec((rb, D), lambda i: (i, 0)),
            pl.BlockSpec((rb, D), lambda i: (i, 0)),
            pl.BlockSpec((rb, D), lambda i: (i, 0)),
            pl.BlockSpec(Wa.shape, lambda i: (0, 0)),
            pl.BlockSpec(Wb.shape, lambda i: (0, 0)),
            pl.BlockSpec(Wc.shape, lambda i: (0, 0)),
            pl.BlockSpec((1, h), lambda i: (0, 0)),
            pl.BlockSpec(Wlin.shape, lambda i: (0, 0)),
            pl.BlockSpec((1, h), lambda i: (0, 0)),
        ],
        out_specs=(
            pl.BlockSpec((rb, h), lambda i: (i, 0)),
            pl.BlockSpec((8, h), lambda i: (0, 0)),
        ),
        out_shape=(jax.ShapeDtypeStruct((N, h), F32),
                   jax.ShapeDtypeStruct((8, h), F32)),
    )(x, cnt, s4, q4, mx4, mn4, Wa, Wb, Wc, bpost, Wlin, blin)


def _var_body(h_ref, st_ref, v_ref):
    i = pl.program_id(0)
    mu = st_ref[0:1] / N
    dd = h_ref[...] - mu
    vs = jnp.sum(dd * dd, axis=0, keepdims=True)
    vv = jnp.concatenate([vs, jnp.zeros((7, vs.shape[1]), F32)], axis=0)

    @pl.when(i == 0)
    def _():
        v_ref[...] = vv

    @pl.when(i > 0)
    def _():
        v_ref[...] = v_ref[...] + vv


def _varpass(H, stats):
    h = H.shape[1]
    rb = 1000
    return pl.pallas_call(
        _var_body,
        grid=(N // rb,),
        in_specs=[
            pl.BlockSpec((rb, h), lambda i: (i, 0)),
            pl.BlockSpec((8, h), lambda i: (0, 0)),
        ],
        out_specs=pl.BlockSpec((8, h), lambda i: (0, 0)),
        out_shape=jax.ShapeDtypeStruct((8, h), F32),
    )(H, stats)


def _bn_body(h_ref, st_ref, vr_ref, g_ref, b_ref, o_ref):
    mu = st_ref[0:1] / N
    var = vr_ref[0:1] / N
    y = g_ref[...] * (h_ref[...] - mu) / jnp.sqrt(var + 1e-5) + b_ref[...]
    o_ref[...] = jnp.where(y > 0, y, jnp.exp(y) - 1.0)


def _bn_apply(H, stats, g, b):
    h = H.shape[1]
    rb = 1000
    vr = _varpass(H, stats)
    return pl.pallas_call(
        _bn_body,
        grid=(N // rb,),
        in_specs=[
            pl.BlockSpec((rb, h), lambda i: (i, 0)),
            pl.BlockSpec((8, h), lambda i: (0, 0)),
            pl.BlockSpec((8, h), lambda i: (0, 0)),
            pl.BlockSpec((1, h), lambda i: (0, 0)),
            pl.BlockSpec((1, h), lambda i: (0, 0)),
        ],
        out_specs=pl.BlockSpec((rb, h), lambda i: (i, 0)),
        out_shape=jax.ShapeDtypeStruct((N, h), F32),
    )(H, stats, vr, g.reshape(1, -1), b.reshape(1, -1))


def _bn_head_body(h_ref, st_ref, vr_ref, g_ref, b_ref, Wo_ref, bo_ref, o_ref):
    mu = st_ref[0:1] / N
    var = vr_ref[0:1] / N
    y = g_ref[...] * (h_ref[...] - mu) / jnp.sqrt(var + 1e-5) + b_ref[...]
    y = jnp.where(y > 0, y, jnp.exp(y) - 1.0)
    o_ref[...] = jnp.dot(y, Wo_ref[...], preferred_element_type=F32, precision=lax.Precision.HIGHEST) \
        + bo_ref[...]


def _bn_head(H, stats, g, b, Wout, bout):
    h = H.shape[1]
    rb = 1000
    vr = _varpass(H, stats)
    return pl.pallas_call(
        _bn_head_body,
        grid=(N // rb,),
        in_specs=[
            pl.BlockSpec((rb, h), lambda i: (i, 0)),
            pl.BlockSpec((8, h), lambda i: (0, 0)),
            pl.BlockSpec((8, h), lambda i: (0, 0)),
            pl.BlockSpec((1, h), lambda i: (0, 0)),
            pl.BlockSpec((1, h), lambda i: (0, 0)),
            pl.BlockSpec((h, 1), lambda i: (0, 0)),
            pl.BlockSpec((1, 1), lambda i: (0, 0)),
        ],
        out_specs=pl.BlockSpec((rb, 1), lambda i: (i, 0)),
        out_shape=jax.ShapeDtypeStruct((N, 1), F32),
    )(H, stats, vr, g.reshape(1, -1), b.reshape(1, -1), Wout,
      bout.reshape(1, 1))


def _layer(x, ea, bins, We, be, Wpre, bpre, Wpost, bpost,
           Wlin, blin):
    se, sd, ss, counts, cnt = bins
    d = x.shape[1]
    Wd, Ws, Wpe = Wpre[:d], Wpre[d:2 * d], Wpre[2 * d:]
    Wc, cb = _wprep(We, Wpe, be, bpre)
    xd, xs = _xprep(x, Wd, Ws)
    ee = _eeprep(ea, Wc, cb)
    s4, q4, mx4, mn4 = _scmain(se, sd, ss, counts, xd, xs, ee)
    cnt2 = cnt[:N].reshape(N, 1)
    Wa = Wpost[:d + 4 * d]
    Wb = Wpost[d + 4 * d:d + 8 * d]
    Wcg = Wpost[d + 8 * d:d + 12 * d]
    return _post(x, cnt2, s4, q4, mx4, mn4, Wa, Wb, Wcg,
                 bpost.reshape(1, -1), Wlin, blin.reshape(1, -1))


def kernel(x, edge_index, edge_attr, We1, be1, Wpre1, bpre1, Wpost1, bpost1,
           Wlin1, blin1, g1, bb1, We2, be2, Wpre2, bpre2, Wpost2, bpost2,
           Wlin2, blin2, g2, bb2, Wout, bout):
    src = edge_index[0]
    dst = edge_index[1]
    e1, se, sd, ss, counts, cnt = _binning(dst, src)
    bins = (se, sd, ss, counts, cnt)
    H1, st1 = _layer(x, edge_attr, bins, We1, be1, Wpre1, bpre1,
                     Wpost1, bpost1, Wlin1, blin1)
    h1 = _bn_apply(H1, st1, g1, bb1)
    H2, st2 = _layer(h1, edge_attr, bins, We2, be2, Wpre2, bpre2,
                     Wpost2, bpost2, Wlin2, blin2)
    return _bn_head(H2, st2, g2, bb2, Wout, bout)


# speedup vs baseline: 2.3723x; 1.0014x over previous
import numpy as np
import jax
import jax.numpy as jnp
from jax import lax
from jax.experimental import pallas as pl
from jax.experimental.pallas import tpu as pltpu
from jax.experimental.pallas import tpu_sc as plsc

AVG_LOG = float(np.mean(np.log(np.arange(1, 31, dtype=np.float64))))
F32 = jnp.float32

N = 10000
E = 160000
D = 256
NW = 32
R = 320
NP = NW * R
NSR = 8
SR = R // NSR
NSB = NW * NSR
K = 64
CH = 1600
FL = 2048
ECAP = E + FL
CH2 = 512
FL2 = 1024
SCAP = E + 2 * FL2

_mesh = plsc.VectorSubcoreMesh(core_axis_name="c", subcore_axis_name="s",
                               num_cores=2, num_subcores=16)
_scparams = pltpu.CompilerParams(needs_layout_passes=False,
                                 use_tc_tiling_on_sc=False)


def _wid():
    return lax.axis_index("s") * 2 + lax.axis_index("c")


def _bin_body(dst_hbm, src_hbm,
              e1_hbm, se_hbm, sd_hbm, ss_hbm, counts_hbm, cnt_hbm,
              dbuf, st_e, cacc, cvec, ebuf2, dv2, sv2, st2e, st2d, st2s,
              sem, sem2):
    w = _wid()
    lo = w * R
    iota = lax.iota(jnp.int32, 16)
    ones = jnp.ones((16,), F32)
    zeros = jnp.zeros((16,), F32)

    def initc(t, _):
        cacc[pl.ds(t * 16, 16)] = zeros
        return 0
    lax.fori_loop(0, R // 16, initc, 0)

    def chunk(ci, carry):
        pltpu.sync_copy(dst_hbm.at[pl.ds(ci * CH, CH)], dbuf)

        def vec(vi, c2):
            wp, off = c2
            d16 = dbuf[pl.ds(vi * 16, 16)]
            eid = ci * CH + vi * 16 + iota
            m = (d16 >= lo) & (d16 < lo + R)
            ld = jnp.clip(d16 - lo, 0, R - 1)
            plsc.addupdate_scatter(cacc, [ld], ones, mask=m)
            plsc.store_compressed(st_e.at[pl.ds(wp, 16)], eid, mask=m)
            npop = plsc.all_reduce_population_count(m)
            if npop.ndim:
                npop = npop[0]
            wp2 = wp + npop
            fl = wp2 >= FL

            @pl.when(fl)
            def _flush():
                pltpu.sync_copy(
                    st_e.at[pl.ds(0, FL)],
                    e1_hbm.at[pl.ds(pl.multiple_of(w * ECAP + off, 8), FL)])
                st_e[pl.ds(0, 16)] = st_e[pl.ds(FL, 16)]

            wp3 = jnp.where(fl, wp2 - FL, wp2)
            off2 = jnp.where(fl, off + FL, off)
            return wp3, off2

        return lax.fori_loop(0, CH // 16, vec, carry)

    wp, off = lax.fori_loop(0, E // CH, chunk,
                            (jnp.int32(0), jnp.int32(0)))
    pltpu.sync_copy(st_e.at[pl.ds(0, FL)],
                    e1_hbm.at[pl.ds(pl.multiple_of(w * ECAP + off, 8), FL)])
    cw = off + wp
    pltpu.sync_copy(cacc, cnt_hbm.at[pl.ds(pl.multiple_of(lo, 8), R)])

    nch = lax.div(cw + (CH2 - 1), jnp.int32(CH2))

    def chunk2(ci, carry):
        cbase = ci * CH2
        pltpu.sync_copy(
            e1_hbm.at[pl.ds(pl.multiple_of(w * ECAP + cbase, 8), CH2)],
            ebuf2)
        for t in range(CH2 // 16):
            mkv = (cbase + t * 16 + iota) < cw
            ebuf2[pl.ds(t * 16, 16)] = jnp.where(
                mkv, ebuf2[pl.ds(t * 16, 16)], 0)
        for g in range(CH2 // 128):
            c0 = pltpu.async_copy(dst_hbm.at[ebuf2.at[pl.ds(g * 128, 128)]],
                                  dv2.at[pl.ds(g * 128, 128)], sem)
            c1 = pltpu.async_copy(src_hbm.at[ebuf2.at[pl.ds(g * 128, 128)]],
                                  sv2.at[pl.ds(g * 128, 128)], sem2)
            c0.wait()
            c1.wait()

        def vec2(vi, c2):
            wps = list(c2)
            valid = (cbase + vi * 16 + iota) < cw
            e16 = ebuf2[pl.ds(vi * 16, 16)]
            d16 = dv2[pl.ds(vi * 16, 16)]
            s16 = sv2[pl.ds(vi * 16, 16)]
            for s in range(NSR):
                slo = lo + s * SR
                m = valid & (d16 >= slo) & (d16 < slo + SR)
                plsc.store_compressed(st2e.at[s, pl.ds(wps[s], 16)], e16,
                                      mask=m)
                plsc.store_compressed(st2d.at[s, pl.ds(wps[s], 16)], d16,
                                      mask=m)
                plsc.store_compressed(st2s.at[s, pl.ds(wps[s], 16)], s16,
                                      mask=m)
                npop = plsc.all_reduce_population_count(m)
                if npop.ndim:
                    npop = npop[0]
                wps[s] = wps[s] + npop
            return tuple(wps)

        wps = list(lax.fori_loop(0, CH2 // 16, vec2, tuple(carry[:NSR])))
        offs = list(carry[NSR:])
        for s in range(NSR):
            fl = wps[s] >= FL2

            @pl.when(fl)
            def _flush(s=s, off=offs[s]):
                for st2, bh in ((st2e, se_hbm), (st2d, sd_hbm),
                                (st2s, ss_hbm)):
                    pltpu.sync_copy(
                        st2.at[s, pl.ds(0, FL2)],
                        bh.at[pl.ds(
                            pl.multiple_of((w * NSR + s) * SCAP + off, 8),
                            FL2)])
                    for t in range(CH2 // 16):
                        st2[s, pl.ds(t * 16, 16)] = \
                            st2[s, pl.ds(FL2 + t * 16, 16)]

            wps[s] = jnp.where(fl, wps[s] - FL2, wps[s])
            offs[s] = jnp.where(fl, offs[s] + FL2, offs[s])
        return tuple(wps) + tuple(offs)

    z = jnp.int32(0)
    carry = lax.fori_loop(0, nch, chunk2, (z,) * NSR + (z,) * NSR)
    for s in range(NSR):
        wps, offs = carry[s], carry[NSR + s]
        for st2, bh in ((st2e, se_hbm), (st2d, sd_hbm), (st2s, ss_hbm)):
            pltpu.sync_copy(
                st2.at[s, pl.ds(0, FL2)],
                bh.at[pl.ds(
                    pl.multiple_of((w * NSR + s) * SCAP + offs, 8), FL2)])

            @pl.when(wps > FL2)
            def _flush2(st2=st2, bh=bh, s=s, off2=offs + FL2):
                pltpu.sync_copy(
                    st2.at[s, pl.ds(FL2, CH2)],
                    bh.at[pl.ds(
                        pl.multiple_of((w * NSR + s) * SCAP + off2, 8),
                        CH2)])

        cvec[...] = jnp.full((16,), offs + wps, jnp.int32)
        pltpu.sync_copy(
            cvec.at[pl.ds(0, 8)],
            counts_hbm.at[pl.ds(pl.multiple_of((w * NSR + s) * 8, 8), 8)])


def _binning(dst, src):
    f = pl.kernel(
        _bin_body,
        compiler_params=_scparams,
        out_type=(
            jax.ShapeDtypeStruct((NW * ECAP,), jnp.int32),
            jax.ShapeDtypeStruct((NSB * SCAP,), jnp.int32),
            jax.ShapeDtypeStruct((NSB * SCAP,), jnp.int32),
            jax.ShapeDtypeStruct((NSB * SCAP,), jnp.int32),
            jax.ShapeDtypeStruct((NSB * 8,), jnp.int32),
            jax.ShapeDtypeStruct((NP,), F32),
        ),
        mesh=_mesh,
        scratch_types=[
            pltpu.VMEM((CH,), jnp.int32),
            pltpu.VMEM((FL + 16,), jnp.int32),
            pltpu.VMEM((R,), F32),
            pltpu.VMEM((16,), jnp.int32),
            pltpu.VMEM((CH2,), jnp.int32),
            pltpu.VMEM((CH2,), jnp.int32),
            pltpu.VMEM((CH2,), jnp.int32),
            pltpu.VMEM((NSR, FL2 + CH2 + 16), jnp.int32),
            pltpu.VMEM((NSR, FL2 + CH2 + 16), jnp.int32),
            pltpu.VMEM((NSR, FL2 + CH2 + 16), jnp.int32),
            pltpu.SemaphoreType.DMA,
            pltpu.SemaphoreType.DMA,
        ],
    )
    return f(dst, src)


def _scmain_body(se_hbm, sd_hbm, ss_hbm, counts_hbm, xd_hbm, xs_hbm,
                 ee_hbm,
                 sum_hbm, sq_hbm, mx_hbm, mn_hbm,
                 cbuf, xdloc,
                 rxsA, reeA, evA, dvA, svA,
                 rxsB, reeB, evB, dvB, svB,
                 acc_s, acc_q, acc_mx, acc_mn,
                 semeA, semxA, semeB, semxB):
    w = _wid()
    iota = lax.iota(jnp.int32, 16)
    zeros = jnp.zeros((16,), F32)
    neg = jnp.full((16,), -3.0e38, F32)
    pos = jnp.full((16,), 3.0e38, F32)

    pltpu.sync_copy(counts_hbm, cbuf.at[pl.ds(0, NSB * 8)])

    def subbin(s, _):
        lo_s = w * R + s * SR
        sb = w * NSR + s
        sc = cbuf[pl.ds(sb * 8, 16)][0]
        nblk = lax.div(sc + (K - 1), jnp.int32(K))
        npair = lax.div(nblk + 1, jnp.int32(2))

        def initrow(r, _):
            for v in range(D // 16):
                cs = pl.ds(v * 16, 16)
                acc_s[r, cs] = zeros
                acc_q[r, cs] = zeros
                acc_mx[r, cs] = neg
                acc_mn[r, cs] = pos
            return 0
        lax.fori_loop(0, SR, initrow, 0)

        pltpu.sync_copy(
            xd_hbm.at[pl.ds(pl.multiple_of(lo_s, 8), SR)], xdloc)

        def issue(b, ev, dv, sv, rxs, ree, seme, semx):
            base = b * K
            pltpu.sync_copy(
                se_hbm.at[pl.ds(pl.multiple_of(sb * SCAP + base, 8), K)],
                ev)
            pltpu.sync_copy(
                sd_hbm.at[pl.ds(pl.multiple_of(sb * SCAP + base, 8), K)],
                dv.at[pl.ds(0, K)])
            pltpu.sync_copy(
                ss_hbm.at[pl.ds(pl.multiple_of(sb * SCAP + base, 8), K)],
                sv)
            nval = jnp.minimum(sc - base, K)
            trash = lo_s + SR
            for t in range(K // 16):
                mk = (t * 16 + iota) < nval
                ev[pl.ds(t * 16, 16)] = jnp.where(
                    mk, ev[pl.ds(t * 16, 16)], 0)
                sv[pl.ds(t * 16, 16)] = jnp.where(
                    mk, sv[pl.ds(t * 16, 16)], 0)
                dv[pl.ds(t * 16, 16)] = jnp.where(
                    mk, dv[pl.ds(t * 16, 16)], trash)
            pltpu.async_copy(ee_hbm.at[ev], ree, seme)
            pltpu.async_copy(xs_hbm.at[sv], rxs, semx)

        def consume(b, ev, dv, sv, rxs, ree, seme, semx):
            pltpu.make_async_copy(ee_hbm.at[ev], ree, seme).wait()
            pltpu.make_async_copy(xs_hbm.at[sv], rxs, semx).wait()
            nval = jnp.minimum(sc - b * K, K)
            ngrp = lax.div(nval + 3, jnp.int32(4))

            def edge4(g, _):
                j0 = g * 4
                for u in range(4):
                    j = j0 + u
                    r = dv[pl.ds(j, 16)][0] - lo_s
                    for v in range(D // 16):
                        cs = pl.ds(v * 16, 16)
                        mv = xdloc[r, cs] + rxs[j, cs] + ree[j, cs]
                        plsc.addupdate(acc_s.at[r, cs], mv)
                        plsc.addupdate(acc_q.at[r, cs], mv * mv)
                        acc_mx[r, cs] = jnp.maximum(acc_mx[r, cs], mv)
                        acc_mn[r, cs] = jnp.minimum(acc_mn[r, cs], mv)
                return 0
            lax.fori_loop(0, ngrp, edge4, 0)

        @pl.when(nblk > 0)
        def _():
            issue(jnp.int32(0), evA, dvA, svA, rxsA, reeA, semeA, semxA)

        def pair(p, _):
            b0 = 2 * p
            b1 = 2 * p + 1

            @pl.when(b1 < nblk)
            def _():
                issue(b1, evB, dvB, svB, rxsB, reeB, semeB, semxB)

            consume(b0, evA, dvA, svA, rxsA, reeA, semeA, semxA)

            @pl.when(b1 + 1 < nblk)
            def _():
                issue(b1 + 1, evA, dvA, svA, rxsA, reeA, semeA, semxA)

            @pl.when(b1 < nblk)
            def _():
                consume(b1, evB, dvB, svB, rxsB, reeB, semeB, semxB)
            return 0

        lax.fori_loop(0, npair, pair, 0)

        pltpu.sync_copy(acc_s.at[pl.ds(0, SR)],
                        sum_hbm.at[pl.ds(pl.multiple_of(lo_s, 8), SR)])
        pltpu.sync_copy(acc_q.at[pl.ds(0, SR)],
                        sq_hbm.at[pl.ds(pl.multiple_of(lo_s, 8), SR)])
        pltpu.sync_copy(acc_mx.at[pl.ds(0, SR)],
                        mx_hbm.at[pl.ds(pl.multiple_of(lo_s, 8), SR)])
        pltpu.sync_copy(acc_mn.at[pl.ds(0, SR)],
                        mn_hbm.at[pl.ds(pl.multiple_of(lo_s, 8), SR)])
        return 0

    lax.fori_loop(0, NSR, subbin, 0)


def _scmain(se, sd, ss, counts, xd, xs, ee):
    f = pl.kernel(
        _scmain_body,
        compiler_params=_scparams,
        out_type=(
            jax.ShapeDtypeStruct((NP, D), F32),
            jax.ShapeDtypeStruct((NP, D), F32),
            jax.ShapeDtypeStruct((NP, D), F32),
            jax.ShapeDtypeStruct((NP, D), F32),
        ),
        mesh=_mesh,
        scratch_types=[
            pltpu.VMEM((NSB * 8 + 16,), jnp.int32),
            pltpu.VMEM((SR, D), F32),
            pltpu.VMEM((K, D), F32),
            pltpu.VMEM((K, D), F32),
            pltpu.VMEM((K,), jnp.int32),
            pltpu.VMEM((K + 16,), jnp.int32),
            pltpu.VMEM((K,), jnp.int32),
            pltpu.VMEM((K, D), F32),
            pltpu.VMEM((K, D), F32),
            pltpu.VMEM((K,), jnp.int32),
            pltpu.VMEM((K + 16,), jnp.int32),
            pltpu.VMEM((K,), jnp.int32),
            pltpu.VMEM((SR + 8, D), F32),
            pltpu.VMEM((SR + 8, D), F32),
            pltpu.VMEM((SR + 8, D), F32),
            pltpu.VMEM((SR + 8, D), F32),
            pltpu.SemaphoreType.DMA,
            pltpu.SemaphoreType.DMA,
            pltpu.SemaphoreType.DMA,
            pltpu.SemaphoreType.DMA,
        ],
    )
    return f(se, sd, ss, counts, xd, xs, ee)


def _wprep_body(We_ref, Wpe_ref, be_ref, bpre_ref, Wc_ref, cb_ref):
    Wc_ref[...] = jnp.dot(We_ref[...], Wpe_ref[...],
                          preferred_element_type=F32, precision=lax.Precision.HIGHEST)
    cb_ref[...] = jnp.dot(be_ref[...], Wpe_ref[...],
                          preferred_element_type=F32, precision=lax.Precision.HIGHEST) + bpre_ref[...]


def _wprep(We, Wpe, be, bpre):
    de, d = We.shape
    return pl.pallas_call(
        _wprep_body,
        out_shape=(jax.ShapeDtypeStruct((de, d), F32),
                   jax.ShapeDtypeStruct((1, d), F32)),
    )(We, Wpe, be.reshape(1, -1), bpre.reshape(1, -1))


def _xprep_body(x_ref, Wd_ref, Ws_ref, xd_ref, xs_ref):
    xb = x_ref[...]
    xd_ref[...] = jnp.dot(xb, Wd_ref[...], preferred_element_type=F32, precision=lax.Precision.HIGHEST)
    xs_ref[...] = jnp.dot(xb, Ws_ref[...], preferred_element_type=F32, precision=lax.Precision.HIGHEST)


def _xprep(x, Wd, Ws):
    d = x.shape[1]
    nb = NP // R
    return pl.pallas_call(
        _xprep_body,
        grid=(nb,),
        in_specs=[
            pl.BlockSpec((R, d), lambda i: (i, 0)),
            pl.BlockSpec((d, D), lambda i: (0, 0)),
            pl.BlockSpec((d, D), lambda i: (0, 0)),
        ],
        out_specs=(
            pl.BlockSpec((R, D), lambda i: (i, 0)),
            pl.BlockSpec((R, D), lambda i: (i, 0)),
        ),
        out_shape=(jax.ShapeDtypeStruct((NP, D), F32),
                   jax.ShapeDtypeStruct((NP, D), F32)),
    )(x, Wd, Ws)


def _eeprep_body(ea_ref, Wc_ref, cb_ref, ee_ref):
    ee_ref[...] = jnp.dot(ea_ref[...], Wc_ref[...],
                          preferred_element_type=F32, precision=lax.Precision.HIGHEST) + cb_ref[...]


def _eeprep(ea, Wc, cb):
    de = ea.shape[1]
    eb = 640
    nb = E // eb
    return pl.pallas_call(
        _eeprep_body,
        grid=(nb,),
        in_specs=[
            pl.BlockSpec((eb, de), lambda i: (i, 0)),
            pl.BlockSpec((de, D), lambda i: (0, 0)),
            pl.BlockSpec((1, D), lambda i: (0, 0)),
        ],
        out_specs=pl.BlockSpec((eb, D), lambda i: (i, 0)),
        out_shape=jax.ShapeDtypeStruct((E, D), F32),
    )(ea, Wc, cb)


def _post_body(x_ref, cnt_ref, s_ref, q_ref, mx_ref, mn_ref,
               Wa_ref, Wb_ref, Wc_ref, bpost_ref, Wlin_ref, blin_ref,
               h_ref, st_ref):
    i = pl.program_id(0)
    cnt = cnt_ref[...]
    cntc = jnp.maximum(cnt, 1.0)
    posm = cnt > 0.0
    mean = s_ref[...] / cntc
    msq = q_ref[...] / cntc
    std = jnp.sqrt(jnp.maximum(msq - mean * mean, 0.0) + 1e-5)
    mx = jnp.where(posm, mx_ref[...], 0.0)
    mn = jnp.where(posm, mn_ref[...], 0.0)
    agg = jnp.concatenate([mean, mx, mn, std], axis=1)
    lc = jnp.log(cntc + 1.0)
    amp = lc / AVG_LOG
    att = AVG_LOG / lc
    cat = jnp.concatenate([x_ref[...], agg], axis=1)
    P = jnp.dot(cat, Wa_ref[...], preferred_element_type=F32, precision=lax.Precision.HIGHEST) + bpost_ref[...]
    P = P + amp * jnp.dot(agg, Wb_ref[...], preferred_element_type=F32, precision=lax.Precision.HIGHEST)
    P = P + att * jnp.dot(agg, Wc_ref[...], preferred_element_type=F32, precision=lax.Precision.HIGHEST)
    H = jnp.dot(P, Wlin_ref[...], preferred_element_type=F32, precision=lax.Precision.HIGHEST) + blin_ref[...]
    h_ref[...] = H
    cs = jnp.sum(H, axis=0, keepdims=True)
    st = jnp.concatenate(
        [cs, jnp.zeros((7, cs.shape[1]), F32)], axis=0)

    @pl.when(i == 0)
    def _():
        st_ref[...] = st

    @pl.when(i > 0)
    def _():
        st_ref[...] = st_ref[...] + st


def _post(x, cnt, s4, q4, mx4, mn4, Wa, Wb, Wc, bpost, Wlin, blin):
    d = x.shape[1]
    h = Wlin.shape[1]
    rb = 1000
    nb = N // rb
    return pl.pallas_call(
        _post_body,
        grid=(nb,),
        in_specs=[
            pl.BlockSpec((rb, d), lambda i: (i, 0)),
            pl.BlockSpec((rb, 1), lambda i: (i, 0)),
            pl.BlockSpec((rb, D), lambda i: (i, 0)),
            pl.BlockSpec((rb, D), lambda i: (i, 0)),
            pl.BlockSpec((rb, D), lambda i: (i, 0)),
            pl.BlockSpec((rb, D), lambda i: (i, 0)),
            pl.BlockSpec(Wa.shape, lambda i: (0, 0)),
            pl.BlockSpec(Wb.shape, lambda i: (0, 0)),
            pl.BlockSpec(Wc.shape, lambda i: (0, 0)),
            pl.BlockSpec((1, h), lambda i: (0, 0)),
            pl.BlockSpec(Wlin.shape, lambda i: (0, 0)),
            pl.BlockSpec((1, h), lambda i: (0, 0)),
        ],
        out_specs=(
            pl.BlockSpec((rb, h), lambda i: (i, 0)),
            pl.BlockSpec((8, h), lambda i: (0, 0)),
        ),
        out_shape=(jax.ShapeDtypeStruct((N, h), F32),
                   jax.ShapeDtypeStruct((8, h), F32)),
    )(x, cnt, s4, q4, mx4, mn4, Wa, Wb, Wc, bpost, Wlin, blin)


def _var_body(h_ref, st_ref, v_ref):
    i = pl.program_id(0)
    mu = st_ref[0:1] / N
    dd = h_ref[...] - mu
    vs = jnp.sum(dd * dd, axis=0, keepdims=True)
    vv = jnp.concatenate([vs, jnp.zeros((7, vs.shape[1]), F32)], axis=0)

    @pl.when(i == 0)
    def _():
        v_ref[...] = vv

    @pl.when(i > 0)
    def _():
        v_ref[...] = v_ref[...] + vv


def _varpass(H, stats):
    h = H.shape[1]
    rb = 1000
    return pl.pallas_call(
        _var_body,
        grid=(N // rb,),
        in_specs=[
            pl.BlockSpec((rb, h), lambda i: (i, 0)),
            pl.BlockSpec((8, h), lambda i: (0, 0)),
        ],
        out_specs=pl.BlockSpec((8, h), lambda i: (0, 0)),
        out_shape=jax.ShapeDtypeStruct((8, h), F32),
    )(H, stats)


def _bn_body(h_ref, st_ref, vr_ref, g_ref, b_ref, o_ref):
    mu = st_ref[0:1] / N
    var = vr_ref[0:1] / N
    y = g_ref[...] * (h_ref[...] - mu) / jnp.sqrt(var + 1e-5) + b_ref[...]
    o_ref[...] = jnp.where(y > 0, y, jnp.exp(y) - 1.0)


def _bn_apply(H, stats, g, b):
    h = H.shape[1]
    rb = 1000
    vr = _varpass(H, stats)
    return pl.pallas_call(
        _bn_body,
        grid=(N // rb,),
        in_specs=[
            pl.BlockSpec((rb, h), lambda i: (i, 0)),
            pl.BlockSpec((8, h), lambda i: (0, 0)),
            pl.BlockSpec((8, h), lambda i: (0, 0)),
            pl.BlockSpec((1, h), lambda i: (0, 0)),
            pl.BlockSpec((1, h), lambda i: (0, 0)),
        ],
        out_specs=pl.BlockSpec((rb, h), lambda i: (i, 0)),
        out_shape=jax.ShapeDtypeStruct((N, h), F32),
    )(H, stats, vr, g.reshape(1, -1), b.reshape(1, -1))


def _bn_head_body(h_ref, st_ref, vr_ref, g_ref, b_ref, Wo_ref, bo_ref, o_ref):
    mu = st_ref[0:1] / N
    var = vr_ref[0:1] / N
    y = g_ref[...] * (h_ref[...] - mu) / jnp.sqrt(var + 1e-5) + b_ref[...]
    y = jnp.where(y > 0, y, jnp.exp(y) - 1.0)
    o_ref[...] = jnp.dot(y, Wo_ref[...], preferred_element_type=F32, precision=lax.Precision.HIGHEST) \
        + bo_ref[...]


def _bn_head(H, stats, g, b, Wout, bout):
    h = H.shape[1]
    rb = 1000
    vr = _varpass(H, stats)
    return pl.pallas_call(
        _bn_head_body,
        grid=(N // rb,),
        in_specs=[
            pl.BlockSpec((rb, h), lambda i: (i, 0)),
            pl.BlockSpec((8, h), lambda i: (0, 0)),
            pl.BlockSpec((8, h), lambda i: (0, 0)),
            pl.BlockSpec((1, h), lambda i: (0, 0)),
            pl.BlockSpec((1, h), lambda i: (0, 0)),
            pl.BlockSpec((h, 1), lambda i: (0, 0)),
            pl.BlockSpec((1, 1), lambda i: (0, 0)),
        ],
        out_specs=pl.BlockSpec((rb, 1), lambda i: (i, 0)),
        out_shape=jax.ShapeDtypeStruct((N, 1), F32),
    )(H, stats, vr, g.reshape(1, -1), b.reshape(1, -1), Wout,
      bout.reshape(1, 1))


def _layer(x, ea, bins, We, be, Wpre, bpre, Wpost, bpost,
           Wlin, blin):
    se, sd, ss, counts, cnt = bins
    d = x.shape[1]
    Wd, Ws, Wpe = Wpre[:d], Wpre[d:2 * d], Wpre[2 * d:]
    Wc, cb = _wprep(We, Wpe, be, bpre)
    xd, xs = _xprep(x, Wd, Ws)
    ee = _eeprep(ea, Wc, cb)
    s4, q4, mx4, mn4 = _scmain(se, sd, ss, counts, xd, xs, ee)
    cnt2 = cnt[:N].reshape(N, 1)
    Wa = Wpost[:d + 4 * d]
    Wb = Wpost[d + 4 * d:d + 8 * d]
    Wcg = Wpost[d + 8 * d:d + 12 * d]
    return _post(x, cnt2, s4, q4, mx4, mn4, Wa, Wb, Wcg,
                 bpost.reshape(1, -1), Wlin, blin.reshape(1, -1))


def kernel(x, edge_index, edge_attr, We1, be1, Wpre1, bpre1, Wpost1, bpost1,
           Wlin1, blin1, g1, bb1, We2, be2, Wpre2, bpre2, Wpost2, bpost2,
           Wlin2, blin2, g2, bb2, Wout, bout):
    src = edge_index[0]
    dst = edge_index[1]
    e1, se, sd, ss, counts, cnt = _binning(dst, src)
    bins = (se, sd, ss, counts, cnt)
    H1, st1 = _layer(x, edge_attr, bins, We1, be1, Wpre1, bpre1,
                     Wpost1, bpost1, Wlin1, blin1)
    h1 = _bn_apply(H1, st1, g1, bb1)
    H2, st2 = _layer(h1, edge_attr, bins, We2, be2, Wpre2, bpre2,
                     Wpost2, bpost2, Wlin2, blin2)
    return _bn_head(H2, st2, g2, bb2, Wout, bout)


# speedup vs baseline: 2.8816x; 1.2147x over previous
import numpy as np
import jax
import jax.numpy as jnp
from jax import lax
from jax.experimental import pallas as pl
from jax.experimental.pallas import tpu as pltpu
from jax.experimental.pallas import tpu_sc as plsc

AVG_LOG = float(np.mean(np.log(np.arange(1, 31, dtype=np.float64))))
F32 = jnp.float32

N = 10000
E = 160000
D = 256
NW = 32
R = 320
NP = NW * R
NSR = 8
SR = R // NSR
NSB = NW * NSR
K = 64
CH = 1600
FL = 2048
ECAP = E + FL
CH2 = 512
FL2 = 1024
SCAP = E + 2 * FL2

_mesh = plsc.VectorSubcoreMesh(core_axis_name="c", subcore_axis_name="s",
                               num_cores=2, num_subcores=16)
_scparams = pltpu.CompilerParams(needs_layout_passes=False,
                                 use_tc_tiling_on_sc=False)
_scparams_tc = pltpu.CompilerParams(needs_layout_passes=False)


def _wid():
    return lax.axis_index("s") * 2 + lax.axis_index("c")


def _bin_body(dst_hbm, src_hbm,
              e1_hbm, se_hbm, sd_hbm, ss_hbm, counts_hbm, cnt_hbm,
              dbuf, st_e, cacc, cvec, ebuf2, dv2, sv2, st2e, st2d, st2s,
              sem, sem2):
    w = _wid()
    lo = w * R
    iota = lax.iota(jnp.int32, 16)
    ones = jnp.ones((16,), F32)
    zeros = jnp.zeros((16,), F32)

    def initc(t, _):
        cacc[pl.ds(t * 16, 16)] = zeros
        return 0
    lax.fori_loop(0, R // 16, initc, 0)

    def chunk(ci, carry):
        pltpu.sync_copy(dst_hbm.at[pl.ds(ci * CH, CH)], dbuf)

        def vec(vi, c2):
            wp, off = c2
            d16 = dbuf[pl.ds(vi * 16, 16)]
            eid = ci * CH + vi * 16 + iota
            m = (d16 >= lo) & (d16 < lo + R)
            ld = jnp.clip(d16 - lo, 0, R - 1)
            plsc.addupdate_scatter(cacc, [ld], ones, mask=m)
            plsc.store_compressed(st_e.at[pl.ds(wp, 16)], eid, mask=m)
            npop = plsc.all_reduce_population_count(m)
            if npop.ndim:
                npop = npop[0]
            wp2 = wp + npop
            fl = wp2 >= FL

            @pl.when(fl)
            def _flush():
                pltpu.sync_copy(
                    st_e.at[pl.ds(0, FL)],
                    e1_hbm.at[pl.ds(pl.multiple_of(w * ECAP + off, 8), FL)])
                st_e[pl.ds(0, 16)] = st_e[pl.ds(FL, 16)]

            wp3 = jnp.where(fl, wp2 - FL, wp2)
            off2 = jnp.where(fl, off + FL, off)
            return wp3, off2

        return lax.fori_loop(0, CH // 16, vec, carry)

    wp, off = lax.fori_loop(0, E // CH, chunk,
                            (jnp.int32(0), jnp.int32(0)))
    pltpu.sync_copy(st_e.at[pl.ds(0, FL)],
                    e1_hbm.at[pl.ds(pl.multiple_of(w * ECAP + off, 8), FL)])
    cw = off + wp
    pltpu.sync_copy(cacc, cnt_hbm.at[pl.ds(pl.multiple_of(lo, 8), R)])

    nch = lax.div(cw + (CH2 - 1), jnp.int32(CH2))

    def chunk2(ci, carry):
        cbase = ci * CH2
        pltpu.sync_copy(
            e1_hbm.at[pl.ds(pl.multiple_of(w * ECAP + cbase, 8), CH2)],
            ebuf2)
        for t in range(CH2 // 16):
            mkv = (cbase + t * 16 + iota) < cw
            ebuf2[pl.ds(t * 16, 16)] = jnp.where(
                mkv, ebuf2[pl.ds(t * 16, 16)], 0)
        for g in range(CH2 // 128):
            c0 = pltpu.async_copy(dst_hbm.at[ebuf2.at[pl.ds(g * 128, 128)]],
                                  dv2.at[pl.ds(g * 128, 128)], sem)
            c1 = pltpu.async_copy(src_hbm.at[ebuf2.at[pl.ds(g * 128, 128)]],
                                  sv2.at[pl.ds(g * 128, 128)], sem2)
            c0.wait()
            c1.wait()

        def vec2(vi, c2):
            wps = list(c2)
            valid = (cbase + vi * 16 + iota) < cw
            e16 = ebuf2[pl.ds(vi * 16, 16)]
            d16 = dv2[pl.ds(vi * 16, 16)]
            s16 = sv2[pl.ds(vi * 16, 16)]
            for s in range(NSR):
                slo = lo + s * SR
                m = valid & (d16 >= slo) & (d16 < slo + SR)
                plsc.store_compressed(st2e.at[s, pl.ds(wps[s], 16)], e16,
                                      mask=m)
                plsc.store_compressed(st2d.at[s, pl.ds(wps[s], 16)], d16,
                                      mask=m)
                plsc.store_compressed(st2s.at[s, pl.ds(wps[s], 16)], s16,
                                      mask=m)
                npop = plsc.all_reduce_population_count(m)
                if npop.ndim:
                    npop = npop[0]
                wps[s] = wps[s] + npop
            return tuple(wps)

        wps = list(lax.fori_loop(0, CH2 // 16, vec2, tuple(carry[:NSR])))
        offs = list(carry[NSR:])
        for s in range(NSR):
            fl = wps[s] >= FL2

            @pl.when(fl)
            def _flush(s=s, off=offs[s]):
                for st2, bh in ((st2e, se_hbm), (st2d, sd_hbm),
                                (st2s, ss_hbm)):
                    pltpu.sync_copy(
                        st2.at[s, pl.ds(0, FL2)],
                        bh.at[pl.ds(
                            pl.multiple_of((w * NSR + s) * SCAP + off, 8),
                            FL2)])
                    for t in range(CH2 // 16):
                        st2[s, pl.ds(t * 16, 16)] = \
                            st2[s, pl.ds(FL2 + t * 16, 16)]

            wps[s] = jnp.where(fl, wps[s] - FL2, wps[s])
            offs[s] = jnp.where(fl, offs[s] + FL2, offs[s])
        return tuple(wps) + tuple(offs)

    z = jnp.int32(0)
    carry = lax.fori_loop(0, nch, chunk2, (z,) * NSR + (z,) * NSR)
    for s in range(NSR):
        wps, offs = carry[s], carry[NSR + s]
        for st2, bh in ((st2e, se_hbm), (st2d, sd_hbm), (st2s, ss_hbm)):
            pltpu.sync_copy(
                st2.at[s, pl.ds(0, FL2)],
                bh.at[pl.ds(
                    pl.multiple_of((w * NSR + s) * SCAP + offs, 8), FL2)])

            @pl.when(wps > FL2)
            def _flush2(st2=st2, bh=bh, s=s, off2=offs + FL2):
                pltpu.sync_copy(
                    st2.at[s, pl.ds(FL2, CH2)],
                    bh.at[pl.ds(
                        pl.multiple_of((w * NSR + s) * SCAP + off2, 8),
                        CH2)])

        cvec[...] = jnp.full((16,), offs + wps, jnp.int32)
        pltpu.sync_copy(
            cvec.at[pl.ds(0, 8)],
            counts_hbm.at[pl.ds(pl.multiple_of((w * NSR + s) * 8, 8), 8)])


def _binning(dst, src):
    f = pl.kernel(
        _bin_body,
        compiler_params=_scparams,
        out_type=(
            jax.ShapeDtypeStruct((NW * ECAP,), jnp.int32),
            jax.ShapeDtypeStruct((NSB * SCAP,), jnp.int32),
            jax.ShapeDtypeStruct((NSB * SCAP,), jnp.int32),
            jax.ShapeDtypeStruct((NSB * SCAP,), jnp.int32),
            jax.ShapeDtypeStruct((NSB * 8,), jnp.int32),
            jax.ShapeDtypeStruct((NP,), F32),
        ),
        mesh=_mesh,
        scratch_types=[
            pltpu.VMEM((CH,), jnp.int32),
            pltpu.VMEM((FL + 16,), jnp.int32),
            pltpu.VMEM((R,), F32),
            pltpu.VMEM((16,), jnp.int32),
            pltpu.VMEM((CH2,), jnp.int32),
            pltpu.VMEM((CH2,), jnp.int32),
            pltpu.VMEM((CH2,), jnp.int32),
            pltpu.VMEM((NSR, FL2 + CH2 + 16), jnp.int32),
            pltpu.VMEM((NSR, FL2 + CH2 + 16), jnp.int32),
            pltpu.VMEM((NSR, FL2 + CH2 + 16), jnp.int32),
            pltpu.SemaphoreType.DMA,
            pltpu.SemaphoreType.DMA,
        ],
    )
    return f(dst, src)


def _scmain_body(se_hbm, sd_hbm, ss_hbm, counts_hbm, xd_hbm, xs_hbm,
                 ee_hbm,
                 sum_hbm, sq_hbm, mx_hbm, mn_hbm,
                 cbuf, xdloc,
                 rxsA, reeA, evA, dvA, svA,
                 rxsB, reeB, evB, dvB, svB,
                 acc_s, acc_q, acc_mx, acc_mn,
                 semeA, semxA, semeB, semxB):
    w = _wid()
    iota = lax.iota(jnp.int32, 16)
    zeros = jnp.zeros((16,), F32)
    neg = jnp.full((16,), -3.0e38, F32)
    pos = jnp.full((16,), 3.0e38, F32)

    pltpu.sync_copy(counts_hbm, cbuf.at[pl.ds(0, NSB * 8)])

    def subbin(s, _):
        lo_s = w * R + s * SR
        sb = w * NSR + s
        sc = cbuf[pl.ds(sb * 8, 16)][0]
        nblk = lax.div(sc + (K - 1), jnp.int32(K))
        npair = lax.div(nblk + 1, jnp.int32(2))

        def initrow(r, _):
            for v in range(D // 16):
                cs = pl.ds(v * 16, 16)
                acc_s[r, cs] = zeros
                acc_q[r, cs] = zeros
                acc_mx[r, cs] = neg
                acc_mn[r, cs] = pos
            return 0
        lax.fori_loop(0, SR, initrow, 0)

        pltpu.sync_copy(
            xd_hbm.at[pl.ds(pl.multiple_of(lo_s, 8), SR)], xdloc)

        def issue(b, ev, dv, sv, rxs, ree, seme, semx):
            base = b * K
            pltpu.sync_copy(
                se_hbm.at[pl.ds(pl.multiple_of(sb * SCAP + base, 8), K)],
                ev)
            pltpu.sync_copy(
                sd_hbm.at[pl.ds(pl.multiple_of(sb * SCAP + base, 8), K)],
                dv.at[pl.ds(0, K)])
            pltpu.sync_copy(
                ss_hbm.at[pl.ds(pl.multiple_of(sb * SCAP + base, 8), K)],
                sv)
            nval = jnp.minimum(sc - base, K)
            trash = lo_s + SR
            for t in range(K // 16):
                mk = (t * 16 + iota) < nval
                ev[pl.ds(t * 16, 16)] = jnp.where(
                    mk, ev[pl.ds(t * 16, 16)], 0)
                sv[pl.ds(t * 16, 16)] = jnp.where(
                    mk, sv[pl.ds(t * 16, 16)], 0)
                dv[pl.ds(t * 16, 16)] = jnp.where(
                    mk, dv[pl.ds(t * 16, 16)], trash)
            pltpu.async_copy(ee_hbm.at[ev], ree, seme)
            pltpu.async_copy(xs_hbm.at[sv], rxs, semx)

        def consume(b, ev, dv, sv, rxs, ree, seme, semx):
            pltpu.make_async_copy(ee_hbm.at[ev], ree, seme).wait()
            pltpu.make_async_copy(xs_hbm.at[sv], rxs, semx).wait()
            nval = jnp.minimum(sc - b * K, K)
            ngrp = lax.div(nval + 3, jnp.int32(4))

            def edge4(g, _):
                j0 = g * 4
                for u in range(4):
                    j = j0 + u
                    r = dv[pl.ds(j, 16)][0] - lo_s
                    for v in range(D // 16):
                        cs = pl.ds(v * 16, 16)
                        mv = xdloc[r, cs] + rxs[j, cs] + ree[j, cs]
                        plsc.addupdate(acc_s.at[r, cs], mv)
                        plsc.addupdate(acc_q.at[r, cs], mv * mv)
                        acc_mx[r, cs] = jnp.maximum(acc_mx[r, cs], mv)
                        acc_mn[r, cs] = jnp.minimum(acc_mn[r, cs], mv)
                return 0
            lax.fori_loop(0, ngrp, edge4, 0)

        @pl.when(nblk > 0)
        def _():
            issue(jnp.int32(0), evA, dvA, svA, rxsA, reeA, semeA, semxA)

        def pair(p, _):
            b0 = 2 * p
            b1 = 2 * p + 1

            @pl.when(b1 < nblk)
            def _():
                issue(b1, evB, dvB, svB, rxsB, reeB, semeB, semxB)

            consume(b0, evA, dvA, svA, rxsA, reeA, semeA, semxA)

            @pl.when(b1 + 1 < nblk)
            def _():
                issue(b1 + 1, evA, dvA, svA, rxsA, reeA, semeA, semxA)

            @pl.when(b1 < nblk)
            def _():
                consume(b1, evB, dvB, svB, rxsB, reeB, semeB, semxB)
            return 0

        lax.fori_loop(0, npair, pair, 0)

        pltpu.sync_copy(acc_s.at[pl.ds(0, SR)],
                        sum_hbm.at[pl.ds(pl.multiple_of(lo_s, 8), SR)])
        pltpu.sync_copy(acc_q.at[pl.ds(0, SR)],
                        sq_hbm.at[pl.ds(pl.multiple_of(lo_s, 8), SR)])
        pltpu.sync_copy(acc_mx.at[pl.ds(0, SR)],
                        mx_hbm.at[pl.ds(pl.multiple_of(lo_s, 8), SR)])
        pltpu.sync_copy(acc_mn.at[pl.ds(0, SR)],
                        mn_hbm.at[pl.ds(pl.multiple_of(lo_s, 8), SR)])
        return 0

    lax.fori_loop(0, NSR, subbin, 0)


def _scmain(se, sd, ss, counts, xd, xs, ee):
    f = pl.kernel(
        _scmain_body,
        compiler_params=_scparams_tc,
        out_type=(
            jax.ShapeDtypeStruct((NP, D), F32),
            jax.ShapeDtypeStruct((NP, D), F32),
            jax.ShapeDtypeStruct((NP, D), F32),
            jax.ShapeDtypeStruct((NP, D), F32),
        ),
        mesh=_mesh,
        scratch_types=[
            pltpu.VMEM((NSB * 8 + 16,), jnp.int32),
            pltpu.VMEM((SR, D), F32),
            pltpu.VMEM((K, D), F32),
            pltpu.VMEM((K, D), F32),
            pltpu.VMEM((K,), jnp.int32),
            pltpu.VMEM((K + 16,), jnp.int32),
            pltpu.VMEM((K,), jnp.int32),
            pltpu.VMEM((K, D), F32),
            pltpu.VMEM((K, D), F32),
            pltpu.VMEM((K,), jnp.int32),
            pltpu.VMEM((K + 16,), jnp.int32),
            pltpu.VMEM((K,), jnp.int32),
            pltpu.VMEM((SR + 8, D), F32),
            pltpu.VMEM((SR + 8, D), F32),
            pltpu.VMEM((SR + 8, D), F32),
            pltpu.VMEM((SR + 8, D), F32),
            pltpu.SemaphoreType.DMA,
            pltpu.SemaphoreType.DMA,
            pltpu.SemaphoreType.DMA,
            pltpu.SemaphoreType.DMA,
        ],
    )
    return f(se, sd, ss, counts, xd, xs, ee)


def _wprep_body(We_ref, Wpe_ref, be_ref, bpre_ref, Wc_ref, cb_ref):
    Wc_ref[...] = jnp.dot(We_ref[...], Wpe_ref[...],
                          preferred_element_type=F32, precision=lax.Precision.HIGHEST)
    cb_ref[...] = jnp.dot(be_ref[...], Wpe_ref[...],
                          preferred_element_type=F32, precision=lax.Precision.HIGHEST) + bpre_ref[...]


def _wprep(We, Wpe, be, bpre):
    de, d = We.shape
    return pl.pallas_call(
        _wprep_body,
        out_shape=(jax.ShapeDtypeStruct((de, d), F32),
                   jax.ShapeDtypeStruct((1, d), F32)),
    )(We, Wpe, be.reshape(1, -1), bpre.reshape(1, -1))


def _xprep_body(x_ref, Wd_ref, Ws_ref, xd_ref, xs_ref):
    xb = x_ref[...]
    xd_ref[...] = jnp.dot(xb, Wd_ref[...], preferred_element_type=F32, precision=lax.Precision.HIGHEST)
    xs_ref[...] = jnp.dot(xb, Ws_ref[...], preferred_element_type=F32, precision=lax.Precision.HIGHEST)


def _xprep(x, Wd, Ws):
    d = x.shape[1]
    nb = NP // R
    return pl.pallas_call(
        _xprep_body,
        grid=(nb,),
        in_specs=[
            pl.BlockSpec((R, d), lambda i: (i, 0)),
            pl.BlockSpec((d, D), lambda i: (0, 0)),
            pl.BlockSpec((d, D), lambda i: (0, 0)),
        ],
        out_specs=(
            pl.BlockSpec((R, D), lambda i: (i, 0)),
            pl.BlockSpec((R, D), lambda i: (i, 0)),
        ),
        out_shape=(jax.ShapeDtypeStruct((NP, D), F32),
                   jax.ShapeDtypeStruct((NP, D), F32)),
    )(x, Wd, Ws)


def _eeprep_body(ea_ref, Wc_ref, cb_ref, ee_ref):
    ee_ref[...] = jnp.dot(ea_ref[...], Wc_ref[...],
                          preferred_element_type=F32, precision=lax.Precision.HIGHEST) + cb_ref[...]


def _eeprep(ea, Wc, cb):
    de = ea.shape[1]
    eb = 640
    nb = E // eb
    return pl.pallas_call(
        _eeprep_body,
        grid=(nb,),
        in_specs=[
            pl.BlockSpec((eb, de), lambda i: (i, 0)),
            pl.BlockSpec((de, D), lambda i: (0, 0)),
            pl.BlockSpec((1, D), lambda i: (0, 0)),
        ],
        out_specs=pl.BlockSpec((eb, D), lambda i: (i, 0)),
        out_shape=jax.ShapeDtypeStruct((E, D), F32),
    )(ea, Wc, cb)


def _post_body(x_ref, cnt_ref, s_ref, q_ref, mx_ref, mn_ref,
               Wa_ref, Wb_ref, Wc_ref, bpost_ref, Wlin_ref, blin_ref,
               h_ref, st_ref):
    i = pl.program_id(0)
    cnt = cnt_ref[...]
    cntc = jnp.maximum(cnt, 1.0)
    posm = cnt > 0.0
    mean = s_ref[...] / cntc
    msq = q_ref[...] / cntc
    std = jnp.sqrt(jnp.maximum(msq - mean * mean, 0.0) + 1e-5)
    mx = jnp.where(posm, mx_ref[...], 0.0)
    mn = jnp.where(posm, mn_ref[...], 0.0)
    agg = jnp.concatenate([mean, mx, mn, std], axis=1)
    lc = jnp.log(cntc + 1.0)
    amp = lc / AVG_LOG
    att = AVG_LOG / lc
    cat = jnp.concatenate([x_ref[...], agg], axis=1)
    P = jnp.dot(cat, Wa_ref[...], preferred_element_type=F32, precision=lax.Precision.HIGHEST) + bpost_ref[...]
    P = P + amp * jnp.dot(agg, Wb_ref[...], preferred_element_type=F32, precision=lax.Precision.HIGHEST)
    P = P + att * jnp.dot(agg, Wc_ref[...], preferred_element_type=F32, precision=lax.Precision.HIGHEST)
    H = jnp.dot(P, Wlin_ref[...], preferred_element_type=F32, precision=lax.Precision.HIGHEST) + blin_ref[...]
    h_ref[...] = H
    cs = jnp.sum(H, axis=0, keepdims=True)
    st = jnp.concatenate(
        [cs, jnp.zeros((7, cs.shape[1]), F32)], axis=0)

    @pl.when(i == 0)
    def _():
        st_ref[...] = st

    @pl.when(i > 0)
    def _():
        st_ref[...] = st_ref[...] + st


def _post(x, cnt, s4, q4, mx4, mn4, Wa, Wb, Wc, bpost, Wlin, blin):
    d = x.shape[1]
    h = Wlin.shape[1]
    rb = 1000
    nb = N // rb
    return pl.pallas_call(
        _post_body,
        grid=(nb,),
        in_specs=[
            pl.BlockSpec((rb, d), lambda i: (i, 0)),
            pl.BlockSpec((rb, 1), lambda i: (i, 0)),
            pl.BlockSpec((rb, D), lambda i: (i, 0)),
            pl.BlockSpec((rb, D), lambda i: (i, 0)),
            pl.BlockSpec((rb, D), lambda i: (i, 0)),
            pl.BlockSpec((rb, D), lambda i: (i, 0)),
            pl.BlockSpec(Wa.shape, lambda i: (0, 0)),
            pl.BlockSpec(Wb.shape, lambda i: (0, 0)),
            pl.BlockSpec(Wc.shape, lambda i: (0, 0)),
            pl.BlockSpec((1, h), lambda i: (0, 0)),
            pl.BlockSpec(Wlin.shape, lambda i: (0, 0)),
            pl.BlockSpec((1, h), lambda i: (0, 0)),
        ],
        out_specs=(
            pl.BlockSpec((rb, h), lambda i: (i, 0)),
            pl.BlockSpec((8, h), lambda i: (0, 0)),
        ),
        out_shape=(jax.ShapeDtypeStruct((N, h), F32),
                   jax.ShapeDtypeStruct((8, h), F32)),
    )(x, cnt, s4, q4, mx4, mn4, Wa, Wb, Wc, bpost, Wlin, blin)


def _var_body(h_ref, st_ref, v_ref):
    i = pl.program_id(0)
    mu = st_ref[0:1] / N
    dd = h_ref[...] - mu
    vs = jnp.sum(dd * dd, axis=0, keepdims=True)
    vv = jnp.concatenate([vs, jnp.zeros((7, vs.shape[1]), F32)], axis=0)

    @pl.when(i == 0)
    def _():
        v_ref[...] = vv

    @pl.when(i > 0)
    def _():
        v_ref[...] = v_ref[...] + vv


def _varpass(H, stats):
    h = H.shape[1]
    rb = 1000
    return pl.pallas_call(
        _var_body,
        grid=(N // rb,),
        in_specs=[
            pl.BlockSpec((rb, h), lambda i: (i, 0)),
            pl.BlockSpec((8, h), lambda i: (0, 0)),
        ],
        out_specs=pl.BlockSpec((8, h), lambda i: (0, 0)),
        out_shape=jax.ShapeDtypeStruct((8, h), F32),
    )(H, stats)


def _bn_body(h_ref, st_ref, vr_ref, g_ref, b_ref, o_ref):
    mu = st_ref[0:1] / N
    var = vr_ref[0:1] / N
    y = g_ref[...] * (h_ref[...] - mu) / jnp.sqrt(var + 1e-5) + b_ref[...]
    o_ref[...] = jnp.where(y > 0, y, jnp.exp(y) - 1.0)


def _bn_apply(H, stats, g, b):
    h = H.shape[1]
    rb = 1000
    vr = _varpass(H, stats)
    return pl.pallas_call(
        _bn_body,
        grid=(N // rb,),
        in_specs=[
            pl.BlockSpec((rb, h), lambda i: (i, 0)),
            pl.BlockSpec((8, h), lambda i: (0, 0)),
            pl.BlockSpec((8, h), lambda i: (0, 0)),
            pl.BlockSpec((1, h), lambda i: (0, 0)),
            pl.BlockSpec((1, h), lambda i: (0, 0)),
        ],
        out_specs=pl.BlockSpec((rb, h), lambda i: (i, 0)),
        out_shape=jax.ShapeDtypeStruct((N, h), F32),
    )(H, stats, vr, g.reshape(1, -1), b.reshape(1, -1))


def _bn_head_body(h_ref, st_ref, vr_ref, g_ref, b_ref, Wo_ref, bo_ref, o_ref):
    mu = st_ref[0:1] / N
    var = vr_ref[0:1] / N
    y = g_ref[...] * (h_ref[...] - mu) / jnp.sqrt(var + 1e-5) + b_ref[...]
    y = jnp.where(y > 0, y, jnp.exp(y) - 1.0)
    o_ref[...] = jnp.dot(y, Wo_ref[...], preferred_element_type=F32, precision=lax.Precision.HIGHEST) \
        + bo_ref[...]


def _bn_head(H, stats, g, b, Wout, bout):
    h = H.shape[1]
    rb = 1000
    vr = _varpass(H, stats)
    return pl.pallas_call(
        _bn_head_body,
        grid=(N // rb,),
        in_specs=[
            pl.BlockSpec((rb, h), lambda i: (i, 0)),
            pl.BlockSpec((8, h), lambda i: (0, 0)),
            pl.BlockSpec((8, h), lambda i: (0, 0)),
            pl.BlockSpec((1, h), lambda i: (0, 0)),
            pl.BlockSpec((1, h), lambda i: (0, 0)),
            pl.BlockSpec((h, 1), lambda i: (0, 0)),
            pl.BlockSpec((1, 1), lambda i: (0, 0)),
        ],
        out_specs=pl.BlockSpec((rb, 1), lambda i: (i, 0)),
        out_shape=jax.ShapeDtypeStruct((N, 1), F32),
    )(H, stats, vr, g.reshape(1, -1), b.reshape(1, -1), Wout,
      bout.reshape(1, 1))


def _layer(x, ea, bins, We, be, Wpre, bpre, Wpost, bpost,
           Wlin, blin):
    se, sd, ss, counts, cnt = bins
    d = x.shape[1]
    Wd, Ws, Wpe = Wpre[:d], Wpre[d:2 * d], Wpre[2 * d:]
    Wc, cb = _wprep(We, Wpe, be, bpre)
    xd, xs = _xprep(x, Wd, Ws)
    ee = _eeprep(ea, Wc, cb)
    s4, q4, mx4, mn4 = _scmain(se, sd, ss, counts, xd, xs, ee)
    cnt2 = cnt[:N].reshape(N, 1)
    Wa = Wpost[:d + 4 * d]
    Wb = Wpost[d + 4 * d:d + 8 * d]
    Wcg = Wpost[d + 8 * d:d + 12 * d]
    return _post(x, cnt2, s4, q4, mx4, mn4, Wa, Wb, Wcg,
                 bpost.reshape(1, -1), Wlin, blin.reshape(1, -1))


def kernel(x, edge_index, edge_attr, We1, be1, Wpre1, bpre1, Wpost1, bpost1,
           Wlin1, blin1, g1, bb1, We2, be2, Wpre2, bpre2, Wpost2, bpost2,
           Wlin2, blin2, g2, bb2, Wout, bout):
    src = edge_index[0]
    dst = edge_index[1]
    e1, se, sd, ss, counts, cnt = _binning(dst, src)
    bins = (se, sd, ss, counts, cnt)
    H1, st1 = _layer(x, edge_attr, bins, We1, be1, Wpre1, bpre1,
                     Wpost1, bpost1, Wlin1, blin1)
    h1 = _bn_apply(H1, st1, g1, bb1)
    H2, st2 = _layer(h1, edge_attr, bins, We2, be2, Wpre2, bpre2,
                     Wpost2, bpost2, Wlin2, blin2)
    return _bn_head(H2, st2, g2, bb2, Wout, bout)
